# baseline (speedup 1.0000x reference)
import jax
import jax.numpy as jnp
from jax.experimental import pallas as pl


def _final_pallas(hg, Wc, bc):
    def body(hg_ref, wc_ref, bc_ref, o_ref):
        o_ref[...] = hg_ref[...] @ wc_ref[...] + bc_ref[...][None, :]

    return pl.pallas_call(
        body,
        out_shape=jax.ShapeDtypeStruct((hg.shape[0], Wc.shape[1]), hg.dtype),
    )(hg, Wc, bc)


def kernel(x, edge_index_r0, edge_weight_r0, edge_index_r1, edge_weight_r1,
           graph_ids, W1_r0, b1_r0, W1_r1, b1_r1, W2_r0, b2_r0, W2_r1,
           b2_r1, Wc, bc):
    N = x.shape[0]
    B = 256
    s0 = edge_index_r0[0].astype(jnp.int32)
    d0 = edge_index_r0[1].astype(jnp.int32)
    s1 = edge_index_r1[0].astype(jnp.int32)
    d1 = edge_index_r1[1].astype(jnp.int32)
    g = graph_ids.astype(jnp.int32)

    od0 = jnp.maximum(jnp.bincount(s0, length=N), 1).astype(jnp.float32)
    id0 = jnp.maximum(jnp.bincount(d0, length=N), 1).astype(jnp.float32)
    od1 = jnp.maximum(jnp.bincount(s1, length=N), 1).astype(jnp.float32)
    id1 = jnp.maximum(jnp.bincount(d1, length=N), 1).astype(jnp.float32)
    cnt = jnp.bincount(g, length=B).astype(jnp.float32)
    inv_cnt = 1.0 / jnp.maximum(cnt, 1.0)
    mask = (cnt >= 1.0).astype(jnp.float32)

    xn0 = x * (od0 ** -0.5)[:, None]
    xn1 = x * (od1 ** -0.5)[:, None]
    agg0 = jnp.zeros((N, 2)).at[d0].add(xn0[s0] * edge_weight_r0[:, None])
    agg1 = jnp.zeros((N, 2)).at[d1].add(xn1[s1] * edge_weight_r1[:, None])
    a = jnp.concatenate(
        [agg0 * (id0 ** -0.5)[:, None], agg1 * (id1 ** -0.5)[:, None]], axis=1)
    Wcat = jnp.concatenate([W1_r0, W1_r1], axis=0)
    h1 = jax.nn.relu(a @ Wcat + b1_r0 + b1_r1)

    h1n0 = h1 * (od0 ** -0.5)[:, None]
    h1n1 = h1 * (od1 ** -0.5)[:, None]
    agg2_0 = jnp.zeros((N, 16)).at[d0].add(h1n0[s0])
    agg2_1 = jnp.zeros((N, 16)).at[d1].add(h1n1[s1])
    q0 = (id0 ** -0.5) * inv_cnt[g]
    q1 = (id1 ** -0.5) * inv_cnt[g]
    pooled0 = jax.ops.segment_sum(agg2_0 * q0[:, None], g, num_segments=B)
    pooled1 = jax.ops.segment_sum(agg2_1 * q1[:, None], g, num_segments=B)
    hg = (pooled0 @ W2_r0 + pooled1 @ W2_r1
          + mask[:, None] * (b2_r0 + b2_r1))
    return _final_pallas(hg, Wc, bc)


# speedup vs baseline: 12.8395x; 12.8395x over previous
import functools

import jax
import jax.numpy as jnp
from jax import lax
from jax.experimental import pallas as pl
from jax.experimental.pallas import tpu as pltpu
from jax.experimental.pallas import tpu_sc as plsc

N = 100000
E = 1600000
B = 256
NP = 102400
EP = 1605632
GRP = EP // 1024
GPT = GRP // 32
NPG = 131072

_MESH = plsc.VectorSubcoreMesh(core_axis_name="c", subcore_axis_name="s")


def _zero_fill(buf, n16):
    z = jnp.zeros((16,), jnp.float32)

    def body(i, _):
        buf[pl.ds(i * 16, 16)] = z
        return 0

    lax.fori_loop(0, n16, body, 0)


CH_A = 7


def _pass_a(s0v, d0v, s1v, d1v, gv,
            degs_out, cnth_out,
            h0, h1, h2, h3, cnth, idxb0, idxb1, idxb2, idxb3,
            ones, zbuf, sem):
    c = lax.axis_index("c")
    s = lax.axis_index("s")
    w = s * 2 + c

    _zero_fill(zbuf, 200)
    for i in range(64):
        ones[pl.ds(i * 16, 16)] = jnp.ones((16,), jnp.float32)

    for h in (h0, h1, h2, h3):
        pltpu.sync_copy(zbuf, h.at[pl.ds(s * 6400, 3200)])
        pltpu.sync_copy(zbuf, h.at[pl.ds(s * 6400 + 3200, 3200)])

    @pl.when(s == 0)
    def _():
        pltpu.sync_copy(zbuf.at[pl.ds(0, 512)], cnth)

    plsc.subcore_barrier()

    arrs = (s0v, d0v, s1v, d1v)
    hs = (h0, h1, h2, h3)
    bufs = (idxb0, idxb1, idxb2, idxb3)

    def chunk(k, _):
        base = (w * GPT + k) * 1024
        for i in range(4):
            pltpu.sync_copy(arrs[i].at[pl.ds(base, 1024)], bufs[i])
        cps = [pltpu.async_copy(ones, hs[i].at[bufs[i]], sem, add=True)
               for i in range(4)]
        for cp in cps:
            cp.wait()
        return 0

    lax.fori_loop(0, GPT, chunk, 0)

    for i in range(4):
        pltpu.sync_copy(gv.at[pl.ds((w * 4 + i) * 1024, 1024)], bufs[i])
    cps = [pltpu.async_copy(ones, cnth.at[bufs[i]], sem, add=True)
           for i in range(4)]
    for cp in cps:
        cp.wait()

    plsc.subcore_barrier()

    for t, h in enumerate((h0, h1, h2, h3)):
        off = (c * 4 + t) * NP + s * 6400
        pltpu.sync_copy(h.at[pl.ds(s * 6400, 6400)],
                        degs_out.at[pl.ds(off, 6400)])

    @pl.when(s == 0)
    def _():
        pltpu.sync_copy(cnth, cnth_out.at[pl.ds(c * 512, 512)])


@functools.partial(
    pl.kernel,
    out_type=[jax.ShapeDtypeStruct((8 * NP,), jnp.float32),
              jax.ShapeDtypeStruct((1024,), jnp.float32)],
    mesh=_MESH,
    scratch_types=[
        pltpu.VMEM_SHARED((NP,), jnp.float32),
        pltpu.VMEM_SHARED((NP,), jnp.float32),
        pltpu.VMEM_SHARED((NP,), jnp.float32),
        pltpu.VMEM_SHARED((NP,), jnp.float32),
        pltpu.VMEM_SHARED((512,), jnp.float32),
        pltpu.VMEM((1024,), jnp.int32),
        pltpu.VMEM((1024,), jnp.int32),
        pltpu.VMEM((1024,), jnp.int32),
        pltpu.VMEM((1024,), jnp.int32),
        pltpu.VMEM((1024,), jnp.float32),
        pltpu.VMEM((3200,), jnp.float32),
        pltpu.SemaphoreType.DMA,
    ],
)
def _sc_hist(s0v, d0v, s1v, d1v, gv, degs_out, cnth_out,
             h0, h1, h2, h3, cnth, idxb0, idxb1, idxb2, idxb3,
             ones, zbuf, sem):
    _pass_a(s0v, d0v, s1v, d1v, gv, degs_out, cnth_out,
            h0, h1, h2, h3, cnth, idxb0, idxb1, idxb2, idxb3,
            ones, zbuf, sem)


def _pass_b(s0e, d0e, ew0e, s1e, d1e, ew1e, xnx0, xny0, xnx1, xny1,
            agg_out,
            txs, tys, ax, ay, sidx, didx, ewb, gx, gy, mx, my, zbuf, sem):
    c = lax.axis_index("c")
    s = lax.axis_index("s")
    w = s * 2 + c

    _zero_fill(zbuf, 200)

    for r, (se, de, ewe, tx, ty) in enumerate(
            ((s0e, d0e, ew0e, xnx0, xny0), (s1e, d1e, ew1e, xnx1, xny1))):
        pltpu.sync_copy(tx.at[pl.ds(s * 6400, 6400)],
                        txs.at[pl.ds(s * 6400, 6400)])
        pltpu.sync_copy(ty.at[pl.ds(s * 6400, 6400)],
                        tys.at[pl.ds(s * 6400, 6400)])
        pltpu.sync_copy(zbuf, ax.at[pl.ds(s * 6400, 3200)])
        pltpu.sync_copy(zbuf, ax.at[pl.ds(s * 6400 + 3200, 3200)])
        pltpu.sync_copy(zbuf, ay.at[pl.ds(s * 6400, 3200)])
        pltpu.sync_copy(zbuf, ay.at[pl.ds(s * 6400 + 3200, 3200)])
        plsc.subcore_barrier()

        def chunk(k, _):
            base = (w * GPT + k) * 1024
            pltpu.sync_copy(se.at[pl.ds(base, 1024)], sidx)
            pltpu.sync_copy(de.at[pl.ds(base, 1024)], didx)
            pltpu.sync_copy(ewe.at[pl.ds(base, 1024)], ewb)
            g0 = pltpu.async_copy(txs.at[sidx], gx, sem)
            g1 = pltpu.async_copy(tys.at[sidx], gy, sem)
            g0.wait()
            g1.wait()

            def mul(l, _):
                sl = pl.ds(l * 16, 16)
                ew16 = ewb[sl]
                mx[sl] = gx[sl] * ew16
                my[sl] = gy[sl] * ew16
                return 0

            lax.fori_loop(0, 64, mul, 0)
            c0 = pltpu.async_copy(mx, ax.at[didx], sem, add=True)
            c1 = pltpu.async_copy(my, ay.at[didx], sem, add=True)
            c0.wait()
            c1.wait()
            return 0

        lax.fori_loop(0, GPT, chunk, 0)
        plsc.subcore_barrier()

        for p, acc in ((0, ax), (1, ay)):
            off = (c * 4 + r * 2 + p) * NP + s * 6400
            pltpu.sync_copy(acc.at[pl.ds(s * 6400, 6400)],
                            agg_out.at[pl.ds(off, 6400)])
        plsc.subcore_barrier()


@functools.partial(
    pl.kernel,
    out_type=jax.ShapeDtypeStruct((8 * NP,), jnp.float32),
    mesh=_MESH,
    scratch_types=[
        pltpu.VMEM_SHARED((NP,), jnp.float32),
        pltpu.VMEM_SHARED((NP,), jnp.float32),
        pltpu.VMEM_SHARED((NP,), jnp.float32),
        pltpu.VMEM_SHARED((NP,), jnp.float32),
        pltpu.VMEM((1024,), jnp.int32),
        pltpu.VMEM((1024,), jnp.int32),
        pltpu.VMEM((1024,), jnp.float32),
        pltpu.VMEM((1024,), jnp.float32),
        pltpu.VMEM((1024,), jnp.float32),
        pltpu.VMEM((1024,), jnp.float32),
        pltpu.VMEM((1024,), jnp.float32),
        pltpu.VMEM((3200,), jnp.float32),
        pltpu.SemaphoreType.DMA,
    ],
)
def _sc_layer1(s0e, d0e, ew0e, s1e, d1e, ew1e, xnx0, xny0, xnx1, xny1,
               agg_out, txs, tys, ax, ay, sidx, didx, ewb, gx, gy, mx, my,
               zbuf, sem):
    _pass_b(s0e, d0e, ew0e, s1e, d1e, ew1e, xnx0, xny0, xnx1, xny1,
            agg_out, txs, tys, ax, ay, sidx, didx, ewb, gx, gy, mx, my,
            zbuf, sem)


def _pass_c(s0e, d0e, s1e, d1e, h1n0, h1n1,
            agg_out,
            acc, sidx0, didx0, rows0, sidx1, didx1, rows1, semg, sems):
    c = lax.axis_index("c")
    s = lax.axis_index("s")
    w = s * 2 + c
    gpt2 = 2 * GPT

    def zrows(i, _):
        rows0[i] = jnp.zeros((16,), jnp.float32)
        return 0

    for r, (se, de, tab) in enumerate(((s0e, d0e, h1n0), (s1e, d1e, h1n1))):
        lax.fori_loop(0, 512, zrows, 0)
        for i in range(12):
            pltpu.sync_copy(rows0, acc.at[pl.ds(s * 6400 + i * 512, 512), :])
        pltpu.sync_copy(rows0.at[pl.ds(0, 256), :],
                        acc.at[pl.ds(s * 6400 + 6144, 256), :])
        plsc.subcore_barrier()

        def chunk(k, _):
            base = (w * gpt2 + 2 * k) * 512
            pltpu.sync_copy(se.at[pl.ds(base, 512)], sidx0)
            pltpu.sync_copy(de.at[pl.ds(base, 512)], didx0)
            g0 = pltpu.async_copy(tab.at[sidx0], rows0, semg)
            pltpu.sync_copy(se.at[pl.ds(base + 512, 512)], sidx1)
            pltpu.sync_copy(de.at[pl.ds(base + 512, 512)], didx1)
            g1 = pltpu.async_copy(tab.at[sidx1], rows1, semg)
            g0.wait()
            c0 = pltpu.async_copy(rows0, acc.at[didx0], sems, add=True)
            g1.wait()
            c1 = pltpu.async_copy(rows1, acc.at[didx1], sems, add=True)
            c0.wait()
            c1.wait()
            return 0

        lax.fori_loop(0, gpt2 // 2, chunk, 0)
        plsc.subcore_barrier()
        off = (c * 2 + r) * NP + s * 6400
        pltpu.sync_copy(acc.at[pl.ds(s * 6400, 6400), :],
                        agg_out.at[pl.ds(off, 6400), :])
        plsc.subcore_barrier()


@functools.partial(
    pl.kernel,
    out_type=jax.ShapeDtypeStruct((4 * NP, 16), jnp.float32),
    mesh=_MESH,
    compiler_params=pltpu.CompilerParams(use_tc_tiling_on_sc=False),
    scratch_types=[
        pltpu.VMEM_SHARED((NP, 16), jnp.float32),
        pltpu.VMEM((512,), jnp.int32),
        pltpu.VMEM((512,), jnp.int32),
        pltpu.VMEM((512, 16), jnp.float32),
        pltpu.VMEM((512,), jnp.int32),
        pltpu.VMEM((512,), jnp.int32),
        pltpu.VMEM((512, 16), jnp.float32),
        pltpu.SemaphoreType.DMA,
        pltpu.SemaphoreType.DMA,
    ],
)
def _sc_layer2(s0e, d0e, s1e, d1e, h1n0, h1n1, agg_out,
               acc, sidx0, didx0, rows0, sidx1, didx1, rows1,
               semg, sems):
    _pass_c(s0e, d0e, s1e, d1e, h1n0, h1n1, agg_out,
            acc, sidx0, didx0, rows0, sidx1, didx1, rows1, semg, sems)


_HP = jax.lax.Precision.HIGHEST
_GRID = NP // 1024


def _t1_call(degs, xT):
    def body(d_ref, x_ref, xp_ref, odi_ref, idi_ref):
        d = d_ref[...]
        ds = d[0] + d[1]
        od0 = jax.lax.rsqrt(jnp.maximum(ds[0], 1.0))
        id0 = jax.lax.rsqrt(jnp.maximum(ds[1], 1.0))
        od1 = jax.lax.rsqrt(jnp.maximum(ds[2], 1.0))
        id1 = jax.lax.rsqrt(jnp.maximum(ds[3], 1.0))
        xv = x_ref[...]
        xp_ref[0] = xv[0] * od0
        xp_ref[1] = xv[1] * od0
        xp_ref[2] = xv[0] * od1
        xp_ref[3] = xv[1] * od1
        odi_ref[0] = od0
        odi_ref[1] = od1
        idi_ref[0] = id0
        idi_ref[1] = id1

    R = NP // 128
    return pl.pallas_call(
        body,
        grid=(R // 8,),
        in_specs=[
            pl.BlockSpec((2, 4, 8, 128), lambda i: (0, 0, i, 0)),
            pl.BlockSpec((2, 8, 128), lambda i: (0, i, 0)),
        ],
        out_specs=[
            pl.BlockSpec((4, 8, 128), lambda i: (0, i, 0)),
            pl.BlockSpec((2, 8, 128), lambda i: (0, i, 0)),
            pl.BlockSpec((2, 8, 128), lambda i: (0, i, 0)),
        ],
        out_shape=[
            jax.ShapeDtypeStruct((4, R, 128), jnp.float32),
            jax.ShapeDtypeStruct((2, R, 128), jnp.float32),
            jax.ShapeDtypeStruct((2, R, 128), jnp.float32),
        ],
    )(degs.reshape(2, 4, R, 128), xT.reshape(2, R, 128))


def _t2_call(at2, idin, odin, Wcat, bsum):
    def body(a_ref, idi_ref, odi_ref, w_ref, b_ref, o0_ref, o1_ref):
        a = a_ref[...]
        asum = a[:, 0, :] + a[:, 1, :]
        idi = idi_ref[...]
        scale = jnp.concatenate(
            [idi[:, 0:1], idi[:, 0:1], idi[:, 1:2], idi[:, 1:2]], axis=1)
        h = jnp.dot(asum * scale, w_ref[...], precision=_HP) + b_ref[...]
        h = jnp.maximum(h, 0.0)
        odi = odi_ref[...]
        o0_ref[...] = h * odi[:, 0:1]
        o1_ref[...] = h * odi[:, 1:2]

    return pl.pallas_call(
        body,
        grid=(_GRID,),
        in_specs=[
            pl.BlockSpec((1024, 2, 4), lambda i: (i, 0, 0)),
            pl.BlockSpec((1024, 2), lambda i: (i, 0)),
            pl.BlockSpec((1024, 2), lambda i: (i, 0)),
            pl.BlockSpec((4, 16), lambda i: (0, 0)),
            pl.BlockSpec((1, 16), lambda i: (0, 0)),
        ],
        out_specs=[
            pl.BlockSpec((1024, 16), lambda i: (i, 0)),
            pl.BlockSpec((1024, 16), lambda i: (i, 0)),
        ],
        out_shape=[
            jax.ShapeDtypeStruct((NP, 16), jnp.float32),
            jax.ShapeDtypeStruct((NP, 16), jnp.float32),
        ],
    )(at2, idin, odin, Wcat, bsum.reshape(1, 16))


def _t3_call(agg2, idin, gcol, cnth, W2_0, W2_1, b2sum, Wc, bc):
    def body(p_ref, idi_ref, g_ref, cnt_ref, w20_ref, w21_ref, b2_ref,
             wc_ref, bc_ref, o_ref, acc0, acc1):
        i = pl.program_id(0)

        @pl.when(i == 0)
        def _():
            acc0[...] = jnp.zeros((256, 16), jnp.float32)
            acc1[...] = jnp.zeros((256, 16), jnp.float32)

        p = p_ref[...]
        m0 = p[0, 0] + p[1, 0]
        m1 = p[0, 1] + p[1, 1]
        cnt = cnt_ref[0, :256] + cnt_ref[1, :256]
        invc = 1.0 / jnp.maximum(cnt, 1.0)
        gids = g_ref[...]
        onehot = (gids == jax.lax.broadcasted_iota(
            jnp.int32, (1, 256), 1)).astype(jnp.float32)
        qc = jnp.dot(onehot, invc[:, None], precision=_HP)
        idi = idi_ref[...]
        q0 = idi[:, 0:1] * qc
        q1 = idi[:, 1:2] * qc
        dn = (((0,), (0,)), ((), ()))
        acc0[...] += jax.lax.dot_general(onehot, m0 * q0, dn, precision=_HP)
        acc1[...] += jax.lax.dot_general(onehot, m1 * q1, dn, precision=_HP)

        @pl.when(i == _GRID - 1)
        def _():
            maskg = (cnt >= 1.0).astype(jnp.float32)
            hg = (jnp.dot(acc0[...], w20_ref[...], precision=_HP)
                  + jnp.dot(acc1[...], w21_ref[...], precision=_HP)
                  + maskg[:, None] * b2_ref[...])
            o_ref[...] = jnp.dot(hg, wc_ref[...], precision=_HP) + bc_ref[...]

    return pl.pallas_call(
        body,
        grid=(_GRID,),
        in_specs=[
            pl.BlockSpec((2, 2, 1024, 16), lambda i: (0, 0, i, 0)),
            pl.BlockSpec((1024, 2), lambda i: (i, 0)),
            pl.BlockSpec((1024, 1), lambda i: (i, 0)),
            pl.BlockSpec((2, 512), lambda i: (0, 0)),
            pl.BlockSpec((16, 16), lambda i: (0, 0)),
            pl.BlockSpec((16, 16), lambda i: (0, 0)),
            pl.BlockSpec((1, 16), lambda i: (0, 0)),
            pl.BlockSpec((16, 2), lambda i: (0, 0)),
            pl.BlockSpec((1, 2), lambda i: (0, 0)),
        ],
        out_specs=pl.BlockSpec((256, 2), lambda i: (0, 0)),
        out_shape=jax.ShapeDtypeStruct((256, 2), jnp.float32),
        scratch_shapes=[
            pltpu.VMEM((256, 16), jnp.float32),
            pltpu.VMEM((256, 16), jnp.float32),
        ],
    )(agg2, idin, gcol, cnth, W2_0, W2_1, b2sum.reshape(1, 16),
      Wc, bc.reshape(1, 2))


def kernel(x, edge_index_r0, edge_weight_r0, edge_index_r1, edge_weight_r1,
           graph_ids, W1_r0, b1_r0, W1_r1, b1_r1, W2_r0, b2_r0, W2_r1,
           b2_r1, Wc, bc):
    s0 = edge_index_r0[0].astype(jnp.int32)
    d0 = edge_index_r0[1].astype(jnp.int32)
    s1 = edge_index_r1[0].astype(jnp.int32)
    d1 = edge_index_r1[1].astype(jnp.int32)
    g = graph_ids.astype(jnp.int32)
    gpad = jnp.concatenate([g, jnp.full((NPG - N,), 256, jnp.int32)])
    epad = jnp.full((EP - E,), N, jnp.int32)
    ewpad = jnp.zeros((EP - E,), jnp.float32)
    s0e = jnp.concatenate([s0, epad])
    d0e = jnp.concatenate([d0, epad])
    s1e = jnp.concatenate([s1, epad])
    d1e = jnp.concatenate([d1, epad])
    ew0e = jnp.concatenate([edge_weight_r0, ewpad])
    ew1e = jnp.concatenate([edge_weight_r1, ewpad])

    degs_flat, cnth_flat = _sc_hist(s0e, d0e, s1e, d1e, gpad)
    cnth = cnth_flat.reshape(2, 512)

    xT = jnp.pad(x.T, ((0, 0), (0, NP - N)))
    xplanes, odi, idi = _t1_call(degs_flat.reshape(2, 4, NP), xT)
    xplanes = xplanes.reshape(4, NP)
    odin = odi.reshape(2, NP).T
    idin = idi.reshape(2, NP).T

    aggB = _sc_layer1(s0e, d0e, ew0e, s1e, d1e, ew1e,
                      xplanes[0], xplanes[1], xplanes[2], xplanes[3])

    at2 = aggB.reshape(2, 4, NP).transpose(2, 0, 1)
    Wcat = jnp.concatenate([W1_r0, W1_r1], axis=0)
    h1n0, h1n1 = _t2_call(at2, idin, odin, Wcat, b1_r0 + b1_r1)

    agg2 = _sc_layer2(s0e, d0e, s1e, d1e, h1n0, h1n1)
    agg2 = agg2.reshape(2, 2, NP, 16)

    gcol = jnp.concatenate(
        [g, jnp.full((NP - N,), 256, jnp.int32)]).reshape(NP, 1)
    return _t3_call(agg2, idin, gcol, cnth, W2_r0, W2_r1,
                    b2_r0 + b2_r1, Wc, bc)


# speedup vs baseline: 16.5321x; 1.2876x over previous
import functools

import jax
import jax.numpy as jnp
from jax import lax
from jax.experimental import pallas as pl
from jax.experimental.pallas import tpu as pltpu
from jax.experimental.pallas import tpu_sc as plsc

N = 100000
E = 1600000
B = 256
NP = 102400
EP = 1605632
GRP = EP // 1024
GPT = GRP // 32
NPG = 131072

_MESH = plsc.VectorSubcoreMesh(core_axis_name="c", subcore_axis_name="s")


def _zero_fill(buf, n16):
    z = jnp.zeros((16,), jnp.float32)

    def body(i, _):
        buf[pl.ds(i * 16, 16)] = z
        return 0

    lax.fori_loop(0, n16, body, 0)


CH_A = 7


def _pass_a(s0v, d0v, s1v, d1v, gv,
            degs_out, cnth_out,
            h0, h1, h2, h3, cnth, idxb0, idxb1, idxb2, idxb3,
            ones, zbuf, sem):
    c = lax.axis_index("c")
    s = lax.axis_index("s")
    w = s * 2 + c

    _zero_fill(zbuf, 200)
    for i in range(64):
        ones[pl.ds(i * 16, 16)] = jnp.ones((16,), jnp.float32)

    for h in (h0, h1, h2, h3):
        pltpu.sync_copy(zbuf, h.at[pl.ds(s * 6400, 3200)])
        pltpu.sync_copy(zbuf, h.at[pl.ds(s * 6400 + 3200, 3200)])

    @pl.when(s == 0)
    def _():
        pltpu.sync_copy(zbuf.at[pl.ds(0, 512)], cnth)

    plsc.subcore_barrier()

    arrs = (s0v, d0v, s1v, d1v)
    hs = (h0, h1, h2, h3)
    bufs = (idxb0, idxb1, idxb2, idxb3)

    def chunk(k, _):
        base = (w * GPT + k) * 1024
        for i in range(4):
            pltpu.sync_copy(arrs[i].at[pl.ds(base, 1024)], bufs[i])
        cps = [pltpu.async_copy(ones, hs[i].at[bufs[i]], sem, add=True)
               for i in range(4)]
        for cp in cps:
            cp.wait()
        return 0

    lax.fori_loop(0, GPT, chunk, 0)

    for i in range(4):
        pltpu.sync_copy(gv.at[pl.ds((w * 4 + i) * 1024, 1024)], bufs[i])
    cps = [pltpu.async_copy(ones, cnth.at[bufs[i]], sem, add=True)
           for i in range(4)]
    for cp in cps:
        cp.wait()

    plsc.subcore_barrier()

    for t, h in enumerate((h0, h1, h2, h3)):
        off = (c * 4 + t) * NP + s * 6400
        pltpu.sync_copy(h.at[pl.ds(s * 6400, 6400)],
                        degs_out.at[pl.ds(off, 6400)])

    @pl.when(s == 0)
    def _():
        pltpu.sync_copy(cnth, cnth_out.at[pl.ds(c * 512, 512)])


@functools.partial(
    pl.kernel,
    out_type=[jax.ShapeDtypeStruct((8 * NP,), jnp.float32),
              jax.ShapeDtypeStruct((1024,), jnp.float32)],
    mesh=_MESH,
    scratch_types=[
        pltpu.VMEM_SHARED((NP,), jnp.float32),
        pltpu.VMEM_SHARED((NP,), jnp.float32),
        pltpu.VMEM_SHARED((NP,), jnp.float32),
        pltpu.VMEM_SHARED((NP,), jnp.float32),
        pltpu.VMEM_SHARED((512,), jnp.float32),
        pltpu.VMEM((1024,), jnp.int32),
        pltpu.VMEM((1024,), jnp.int32),
        pltpu.VMEM((1024,), jnp.int32),
        pltpu.VMEM((1024,), jnp.int32),
        pltpu.VMEM((1024,), jnp.float32),
        pltpu.VMEM((3200,), jnp.float32),
        pltpu.SemaphoreType.DMA,
    ],
)
def _sc_hist(s0v, d0v, s1v, d1v, gv, degs_out, cnth_out,
             h0, h1, h2, h3, cnth, idxb0, idxb1, idxb2, idxb3,
             ones, zbuf, sem):
    _pass_a(s0v, d0v, s1v, d1v, gv, degs_out, cnth_out,
            h0, h1, h2, h3, cnth, idxb0, idxb1, idxb2, idxb3,
            ones, zbuf, sem)


def _pass_b(s0e, d0e, ew0e, s1e, d1e, ew1e, xnx0, xny0, xnx1, xny1,
            agg_out,
            txs, tys, ax, ay, sidx, didx, ewb, gx, gy, mx, my, zbuf, sem):
    c = lax.axis_index("c")
    s = lax.axis_index("s")
    w = s * 2 + c

    _zero_fill(zbuf, 200)

    for r, (se, de, ewe, tx, ty) in enumerate(
            ((s0e, d0e, ew0e, xnx0, xny0), (s1e, d1e, ew1e, xnx1, xny1))):
        pltpu.sync_copy(tx.at[pl.ds(s * 6400, 6400)],
                        txs.at[pl.ds(s * 6400, 6400)])
        pltpu.sync_copy(ty.at[pl.ds(s * 6400, 6400)],
                        tys.at[pl.ds(s * 6400, 6400)])
        pltpu.sync_copy(zbuf, ax.at[pl.ds(s * 6400, 3200)])
        pltpu.sync_copy(zbuf, ax.at[pl.ds(s * 6400 + 3200, 3200)])
        pltpu.sync_copy(zbuf, ay.at[pl.ds(s * 6400, 3200)])
        pltpu.sync_copy(zbuf, ay.at[pl.ds(s * 6400 + 3200, 3200)])
        plsc.subcore_barrier()

        def chunk(k, _):
            base = (w * GPT + k) * 1024
            pltpu.sync_copy(se.at[pl.ds(base, 1024)], sidx)
            pltpu.sync_copy(de.at[pl.ds(base, 1024)], didx)
            pltpu.sync_copy(ewe.at[pl.ds(base, 1024)], ewb)
            g0 = pltpu.async_copy(txs.at[sidx], gx, sem)
            g1 = pltpu.async_copy(tys.at[sidx], gy, sem)
            g0.wait()
            g1.wait()

            def mul(l, _):
                sl = pl.ds(l * 16, 16)
                ew16 = ewb[sl]
                mx[sl] = gx[sl] * ew16
                my[sl] = gy[sl] * ew16
                return 0

            lax.fori_loop(0, 64, mul, 0)
            c0 = pltpu.async_copy(mx, ax.at[didx], sem, add=True)
            c1 = pltpu.async_copy(my, ay.at[didx], sem, add=True)
            c0.wait()
            c1.wait()
            return 0

        lax.fori_loop(0, GPT, chunk, 0)
        plsc.subcore_barrier()

        for p, acc in ((0, ax), (1, ay)):
            off = (c * 4 + r * 2 + p) * NP + s * 6400
            pltpu.sync_copy(acc.at[pl.ds(s * 6400, 6400)],
                            agg_out.at[pl.ds(off, 6400)])
        plsc.subcore_barrier()


@functools.partial(
    pl.kernel,
    out_type=jax.ShapeDtypeStruct((8 * NP,), jnp.float32),
    mesh=_MESH,
    scratch_types=[
        pltpu.VMEM_SHARED((NP,), jnp.float32),
        pltpu.VMEM_SHARED((NP,), jnp.float32),
        pltpu.VMEM_SHARED((NP,), jnp.float32),
        pltpu.VMEM_SHARED((NP,), jnp.float32),
        pltpu.VMEM((1024,), jnp.int32),
        pltpu.VMEM((1024,), jnp.int32),
        pltpu.VMEM((1024,), jnp.float32),
        pltpu.VMEM((1024,), jnp.float32),
        pltpu.VMEM((1024,), jnp.float32),
        pltpu.VMEM((1024,), jnp.float32),
        pltpu.VMEM((1024,), jnp.float32),
        pltpu.VMEM((3200,), jnp.float32),
        pltpu.SemaphoreType.DMA,
    ],
)
def _sc_layer1(s0e, d0e, ew0e, s1e, d1e, ew1e, xnx0, xny0, xnx1, xny1,
               agg_out, txs, tys, ax, ay, sidx, didx, ewb, gx, gy, mx, my,
               zbuf, sem):
    _pass_b(s0e, d0e, ew0e, s1e, d1e, ew1e, xnx0, xny0, xnx1, xny1,
            agg_out, txs, tys, ax, ay, sidx, didx, ewb, gx, gy, mx, my,
            zbuf, sem)


def _pass_c(s0e, d0e, s1e, d1e, h1n0, h1n1,
            agg_out,
            acc, sidx0, didx0, rows0, sidx1, didx1, rows1, semg, sems):
    c = lax.axis_index("c")
    s = lax.axis_index("s")
    w = s * 2 + c
    gpt2 = 2 * GPT

    def zrows(i, _):
        rows0[i] = jnp.zeros((16,), jnp.float32)
        return 0

    for r, (se, de, tab) in enumerate(((s0e, d0e, h1n0), (s1e, d1e, h1n1))):
        lax.fori_loop(0, 512, zrows, 0)
        for i in range(12):
            pltpu.sync_copy(rows0, acc.at[pl.ds(s * 6400 + i * 512, 512), :])
        pltpu.sync_copy(rows0.at[pl.ds(0, 256), :],
                        acc.at[pl.ds(s * 6400 + 6144, 256), :])
        plsc.subcore_barrier()

        def chunk(k, _):
            base = (w * gpt2 + 2 * k) * 512
            pltpu.sync_copy(se.at[pl.ds(base, 512)], sidx0)
            pltpu.sync_copy(de.at[pl.ds(base, 512)], didx0)
            g0 = pltpu.async_copy(tab.at[sidx0], rows0, semg)
            pltpu.sync_copy(se.at[pl.ds(base + 512, 512)], sidx1)
            pltpu.sync_copy(de.at[pl.ds(base + 512, 512)], didx1)
            g1 = pltpu.async_copy(tab.at[sidx1], rows1, semg)
            g0.wait()
            c0 = pltpu.async_copy(rows0, acc.at[didx0], sems, add=True)
            g1.wait()
            c1 = pltpu.async_copy(rows1, acc.at[didx1], sems, add=True)
            c0.wait()
            c1.wait()
            return 0

        lax.fori_loop(0, gpt2 // 2, chunk, 0)
        plsc.subcore_barrier()
        off = (c * 2 + r) * NP + s * 6400
        pltpu.sync_copy(acc.at[pl.ds(s * 6400, 6400), :],
                        agg_out.at[pl.ds(off, 6400), :])
        plsc.subcore_barrier()


@functools.partial(
    pl.kernel,
    out_type=jax.ShapeDtypeStruct((4 * NP, 16), jnp.float32),
    mesh=_MESH,
    compiler_params=pltpu.CompilerParams(use_tc_tiling_on_sc=False),
    scratch_types=[
        pltpu.VMEM_SHARED((NP, 16), jnp.float32),
        pltpu.VMEM((512,), jnp.int32),
        pltpu.VMEM((512,), jnp.int32),
        pltpu.VMEM((512, 16), jnp.float32),
        pltpu.VMEM((512,), jnp.int32),
        pltpu.VMEM((512,), jnp.int32),
        pltpu.VMEM((512, 16), jnp.float32),
        pltpu.SemaphoreType.DMA,
        pltpu.SemaphoreType.DMA,
    ],
)
def _sc_layer2(s0e, d0e, s1e, d1e, h1n0, h1n1, agg_out,
               acc, sidx0, didx0, rows0, sidx1, didx1, rows1,
               semg, sems):
    _pass_c(s0e, d0e, s1e, d1e, h1n0, h1n1, agg_out,
            acc, sidx0, didx0, rows0, sidx1, didx1, rows1, semg, sems)


_HP = jax.lax.Precision.HIGHEST
_GRID = NP // 1024


def _t1_call(degs, xT):
    def body(d_ref, x_ref, xp_ref, odi_ref, idi_ref):
        d = d_ref[...]
        ds = d[0] + d[1]
        od0 = jax.lax.rsqrt(jnp.maximum(ds[0], 1.0))
        id0 = jax.lax.rsqrt(jnp.maximum(ds[1], 1.0))
        od1 = jax.lax.rsqrt(jnp.maximum(ds[2], 1.0))
        id1 = jax.lax.rsqrt(jnp.maximum(ds[3], 1.0))
        xv = x_ref[...]
        xp_ref[0] = xv[0] * od0
        xp_ref[1] = xv[1] * od0
        xp_ref[2] = xv[0] * od1
        xp_ref[3] = xv[1] * od1
        odi_ref[0] = od0
        odi_ref[1] = od1
        idi_ref[0] = id0
        idi_ref[1] = id1

    R = NP // 128
    return pl.pallas_call(
        body,
        grid=(R // 8,),
        in_specs=[
            pl.BlockSpec((2, 4, 8, 128), lambda i: (0, 0, i, 0)),
            pl.BlockSpec((2, 8, 128), lambda i: (0, i, 0)),
        ],
        out_specs=[
            pl.BlockSpec((4, 8, 128), lambda i: (0, i, 0)),
            pl.BlockSpec((2, 8, 128), lambda i: (0, i, 0)),
            pl.BlockSpec((2, 8, 128), lambda i: (0, i, 0)),
        ],
        out_shape=[
            jax.ShapeDtypeStruct((4, R, 128), jnp.float32),
            jax.ShapeDtypeStruct((2, R, 128), jnp.float32),
            jax.ShapeDtypeStruct((2, R, 128), jnp.float32),
        ],
    )(degs.reshape(2, 4, R, 128), xT.reshape(2, R, 128))


def _t2_call(aggB, idi, odi, Wcat, bsum):
    def body(a_ref, idi_ref, odi_ref, w_ref, b_ref, o0_ref, o1_ref):
        a = a_ref[...]
        asum = a[0] + a[1]
        idiv = idi_ref[...]
        a4 = asum * jnp.stack(
            [idiv[0], idiv[0], idiv[1], idiv[1]])
        dn = (((0,), (0,)), ((), ()))
        hT = jax.lax.dot_general(w_ref[...], a4, dn, precision=_HP)
        hT = jnp.maximum(hT + b_ref[...], 0.0)
        odiv = odi_ref[...]
        o0_ref[...] = jax.lax.transpose(hT * odiv[0:1, :], (1, 0))
        o1_ref[...] = jax.lax.transpose(hT * odiv[1:2, :], (1, 0))

    return pl.pallas_call(
        body,
        grid=(_GRID,),
        in_specs=[
            pl.BlockSpec((2, 4, 1024), lambda i: (0, 0, i)),
            pl.BlockSpec((2, 1024), lambda i: (0, i)),
            pl.BlockSpec((2, 1024), lambda i: (0, i)),
            pl.BlockSpec((4, 16), lambda i: (0, 0)),
            pl.BlockSpec((16, 1), lambda i: (0, 0)),
        ],
        out_specs=[
            pl.BlockSpec((1024, 16), lambda i: (i, 0)),
            pl.BlockSpec((1024, 16), lambda i: (i, 0)),
        ],
        out_shape=[
            jax.ShapeDtypeStruct((NP, 16), jnp.float32),
            jax.ShapeDtypeStruct((NP, 16), jnp.float32),
        ],
    )(aggB.reshape(2, 4, NP), idi.reshape(2, NP), odi.reshape(2, NP),
      Wcat, bsum.reshape(16, 1))


def _t3_call(agg2, idi, gcol, cnth, W2_0, W2_1, b2sum, Wc, bc):
    def body(p_ref, idi_ref, g_ref, cnt_ref, w20_ref, w21_ref, b2_ref,
             wc_ref, bc_ref, o_ref, acc):
        i = pl.program_id(0)

        @pl.when(i == 0)
        def _():
            acc[...] = jnp.zeros((256, 32), jnp.float32)

        p = p_ref[...]
        idiv = idi_ref[...]
        i0 = jax.lax.transpose(idiv[0:1, :], (1, 0))
        i1 = jax.lax.transpose(idiv[1:2, :], (1, 0))
        m0 = (p[0, 0] + p[1, 0]) * i0
        m1 = (p[0, 1] + p[1, 1]) * i1
        rhs = jnp.concatenate([m0, m1], axis=1).astype(jnp.bfloat16)
        gids = g_ref[...]
        onehot = (gids == jax.lax.broadcasted_iota(
            jnp.int32, (1, 256), 1)).astype(jnp.bfloat16)
        dn = (((0,), (0,)), ((), ()))
        acc[...] += jax.lax.dot_general(
            onehot, rhs, dn, preferred_element_type=jnp.float32)

        @pl.when(i == _GRID - 1)
        def _():
            cnt = cnt_ref[0, :256] + cnt_ref[1, :256]
            invc = (1.0 / jnp.maximum(cnt, 1.0))[:, None]
            maskg = (cnt >= 1.0).astype(jnp.float32)
            av = acc[...]
            hg = (jnp.dot(av[:, :16] * invc, w20_ref[...], precision=_HP)
                  + jnp.dot(av[:, 16:] * invc, w21_ref[...], precision=_HP)
                  + maskg[:, None] * b2_ref[...])
            o_ref[...] = jnp.dot(hg, wc_ref[...], precision=_HP) + bc_ref[...]

    return pl.pallas_call(
        body,
        grid=(_GRID,),
        in_specs=[
            pl.BlockSpec((2, 2, 1024, 16), lambda i: (0, 0, i, 0)),
            pl.BlockSpec((2, 1024), lambda i: (0, i)),
            pl.BlockSpec((1024, 1), lambda i: (i, 0)),
            pl.BlockSpec((2, 512), lambda i: (0, 0)),
            pl.BlockSpec((16, 16), lambda i: (0, 0)),
            pl.BlockSpec((16, 16), lambda i: (0, 0)),
            pl.BlockSpec((1, 16), lambda i: (0, 0)),
            pl.BlockSpec((16, 2), lambda i: (0, 0)),
            pl.BlockSpec((1, 2), lambda i: (0, 0)),
        ],
        out_specs=pl.BlockSpec((256, 2), lambda i: (0, 0)),
        out_shape=jax.ShapeDtypeStruct((256, 2), jnp.float32),
        scratch_shapes=[
            pltpu.VMEM((256, 32), jnp.float32),
        ],
    )(agg2, idi.reshape(2, NP), gcol, cnth, W2_0, W2_1,
      b2sum.reshape(1, 16), Wc, bc.reshape(1, 2))


def kernel(x, edge_index_r0, edge_weight_r0, edge_index_r1, edge_weight_r1,
           graph_ids, W1_r0, b1_r0, W1_r1, b1_r1, W2_r0, b2_r0, W2_r1,
           b2_r1, Wc, bc):
    s0 = edge_index_r0[0].astype(jnp.int32)
    d0 = edge_index_r0[1].astype(jnp.int32)
    s1 = edge_index_r1[0].astype(jnp.int32)
    d1 = edge_index_r1[1].astype(jnp.int32)
    g = graph_ids.astype(jnp.int32)
    gpad = jnp.concatenate([g, jnp.full((NPG - N,), 256, jnp.int32)])
    epad = jnp.full((EP - E,), N, jnp.int32)
    ewpad = jnp.zeros((EP - E,), jnp.float32)
    s0e = jnp.concatenate([s0, epad])
    d0e = jnp.concatenate([d0, epad])
    s1e = jnp.concatenate([s1, epad])
    d1e = jnp.concatenate([d1, epad])
    ew0e = jnp.concatenate([edge_weight_r0, ewpad])
    ew1e = jnp.concatenate([edge_weight_r1, ewpad])

    degs_flat, cnth_flat = _sc_hist(s0e, d0e, s1e, d1e, gpad)
    cnth = cnth_flat.reshape(2, 512)

    xT = jnp.pad(x.T, ((0, 0), (0, NP - N)))
    xplanes, odi, idi = _t1_call(degs_flat.reshape(2, 4, NP), xT)
    xp = xplanes.reshape(4, NP)

    aggB = _sc_layer1(s0e, d0e, ew0e, s1e, d1e, ew1e,
                      xp[0], xp[1], xp[2], xp[3])

    Wcat = jnp.concatenate([W1_r0, W1_r1], axis=0)
    h1n0, h1n1 = _t2_call(aggB, idi, odi, Wcat, b1_r0 + b1_r1)

    agg2 = _sc_layer2(s0e, d0e, s1e, d1e, h1n0, h1n1)
    agg2 = agg2.reshape(2, 2, NP, 16)

    gcol = jnp.concatenate(
        [g, jnp.full((NP - N,), 256, jnp.int32)]).reshape(NP, 1)
    return _t3_call(agg2, idi, gcol, cnth, W2_r0, W2_r1,
                    b2_r0 + b2_r1, Wc, bc)


# speedup vs baseline: 20.3258x; 1.2295x over previous
import functools

import jax
import jax.numpy as jnp
from jax import lax
from jax.experimental import pallas as pl
from jax.experimental.pallas import tpu as pltpu
from jax.experimental.pallas import tpu_sc as plsc

N = 100000
E = 1600000
B = 256
NP = 102400
EP = 1605632
GRP = EP // 1024
GPT = GRP // 32
NPG = 131072

_MESH = plsc.VectorSubcoreMesh(core_axis_name="c", subcore_axis_name="s")


def _zero_fill(buf, n16):
    z = jnp.zeros((16,), jnp.float32)

    def body(i, _):
        buf[pl.ds(i * 16, 16)] = z
        return 0

    lax.fori_loop(0, n16, body, 0)


CH_A = 7


def _pass_a(s0v, d0v, s1v, d1v, gv,
            degs_out, cnth_out,
            h0, h1, h2, h3, cnth, idxb0, idxb1, idxb2, idxb3,
            ones, zbuf, sem):
    c = lax.axis_index("c")
    s = lax.axis_index("s")
    w = s * 2 + c

    _zero_fill(zbuf, 200)
    for i in range(64):
        ones[pl.ds(i * 16, 16)] = jnp.ones((16,), jnp.float32)

    for h in (h0, h1, h2, h3):
        pltpu.sync_copy(zbuf, h.at[pl.ds(s * 6400, 3200)])
        pltpu.sync_copy(zbuf, h.at[pl.ds(s * 6400 + 3200, 3200)])

    @pl.when(s == 0)
    def _():
        pltpu.sync_copy(zbuf.at[pl.ds(0, 512)], cnth)

    plsc.subcore_barrier()

    arrs = (s0v, d0v, s1v, d1v)
    hs = (h0, h1, h2, h3)
    bufs = (idxb0, idxb1, idxb2, idxb3)

    def chunk(k, _):
        base = (w * GPT + k) * 1024
        for i in range(4):
            pltpu.sync_copy(arrs[i].at[pl.ds(base, 1024)], bufs[i])
        cps = [pltpu.async_copy(ones, hs[i].at[bufs[i]], sem, add=True)
               for i in range(4)]
        for cp in cps:
            cp.wait()
        return 0

    lax.fori_loop(0, GPT, chunk, 0)

    for i in range(4):
        pltpu.sync_copy(gv.at[pl.ds((w * 4 + i) * 1024, 1024)], bufs[i])
    cps = [pltpu.async_copy(ones, cnth.at[bufs[i]], sem, add=True)
           for i in range(4)]
    for cp in cps:
        cp.wait()

    plsc.subcore_barrier()

    for t, h in enumerate((h0, h1, h2, h3)):
        off = (c * 4 + t) * NP + s * 6400
        pltpu.sync_copy(h.at[pl.ds(s * 6400, 6400)],
                        degs_out.at[pl.ds(off, 6400)])

    @pl.when(s == 0)
    def _():
        pltpu.sync_copy(cnth, cnth_out.at[pl.ds(c * 512, 512)])


@functools.partial(
    pl.kernel,
    out_type=[jax.ShapeDtypeStruct((8 * NP,), jnp.float32),
              jax.ShapeDtypeStruct((1024,), jnp.float32)],
    mesh=_MESH,
    scratch_types=[
        pltpu.VMEM_SHARED((NP,), jnp.float32),
        pltpu.VMEM_SHARED((NP,), jnp.float32),
        pltpu.VMEM_SHARED((NP,), jnp.float32),
        pltpu.VMEM_SHARED((NP,), jnp.float32),
        pltpu.VMEM_SHARED((512,), jnp.float32),
        pltpu.VMEM((1024,), jnp.int32),
        pltpu.VMEM((1024,), jnp.int32),
        pltpu.VMEM((1024,), jnp.int32),
        pltpu.VMEM((1024,), jnp.int32),
        pltpu.VMEM((1024,), jnp.float32),
        pltpu.VMEM((3200,), jnp.float32),
        pltpu.SemaphoreType.DMA,
    ],
)
def _sc_hist(s0v, d0v, s1v, d1v, gv, degs_out, cnth_out,
             h0, h1, h2, h3, cnth, idxb0, idxb1, idxb2, idxb3,
             ones, zbuf, sem):
    _pass_a(s0v, d0v, s1v, d1v, gv, degs_out, cnth_out,
            h0, h1, h2, h3, cnth, idxb0, idxb1, idxb2, idxb3,
            ones, zbuf, sem)


def _pass_b(s0e, d0e, ew0e, s1e, d1e, ew1e, xnx0, xny0, xnx1, xny1,
            agg_out,
            txs, tys, ax, ay, sidx, didxA, didxB, ewb, gx, gy,
            mxA, myA, mxB, myB, zbuf, semL, semG, semC):
    c = lax.axis_index("c")
    s = lax.axis_index("s")
    w = s * 2 + c
    GB = 3584

    _zero_fill(zbuf, 200)

    for r, (se, de, ewe, tx, ty) in enumerate(
            ((s0e, d0e, ew0e, xnx0, xny0), (s1e, d1e, ew1e, xnx1, xny1))):
        pltpu.sync_copy(tx.at[pl.ds(s * 6400, 6400)],
                        txs.at[pl.ds(s * 6400, 6400)])
        pltpu.sync_copy(ty.at[pl.ds(s * 6400, 6400)],
                        tys.at[pl.ds(s * 6400, 6400)])
        pltpu.sync_copy(zbuf, ax.at[pl.ds(s * 6400, 3200)])
        pltpu.sync_copy(zbuf, ax.at[pl.ds(s * 6400 + 3200, 3200)])
        pltpu.sync_copy(zbuf, ay.at[pl.ds(s * 6400, 3200)])
        pltpu.sync_copy(zbuf, ay.at[pl.ds(s * 6400 + 3200, 3200)])
        plsc.subcore_barrier()

        def one_group(base, didx, mx, my):
            l0 = pltpu.async_copy(se.at[pl.ds(base, GB)], sidx, semL)
            l1 = pltpu.async_copy(de.at[pl.ds(base, GB)], didx, semL)
            l2 = pltpu.async_copy(ewe.at[pl.ds(base, GB)], ewb, semL)
            l0.wait()
            l1.wait()
            l2.wait()
            g0 = pltpu.async_copy(txs.at[sidx], gx, semG)
            g1 = pltpu.async_copy(tys.at[sidx], gy, semG)
            g0.wait()
            g1.wait()

            def mul(l, _):
                for u in range(4):
                    sl = pl.ds((l * 4 + u) * 16, 16)
                    ew16 = ewb[sl]
                    mx[sl] = gx[sl] * ew16
                    my[sl] = gy[sl] * ew16
                return 0

            lax.fori_loop(0, GB // 64, mul, 0)
            c0 = pltpu.async_copy(mx, ax.at[didx], semC, add=True)
            c1 = pltpu.async_copy(my, ay.at[didx], semC, add=True)
            return c0, c1

        def pair(j, _):
            base = (w * 14 + 2 * j) * GB
            cA0, cA1 = one_group(base, didxA, mxA, myA)
            cB0, cB1 = one_group(base + GB, didxB, mxB, myB)
            cA0.wait()
            cA1.wait()
            cB0.wait()
            cB1.wait()
            return 0

        lax.fori_loop(0, 7, pair, 0)
        plsc.subcore_barrier()

        for p, acc in ((0, ax), (1, ay)):
            off = (c * 4 + r * 2 + p) * NP + s * 6400
            pltpu.sync_copy(acc.at[pl.ds(s * 6400, 6400)],
                            agg_out.at[pl.ds(off, 6400)])
        plsc.subcore_barrier()


@functools.partial(
    pl.kernel,
    out_type=jax.ShapeDtypeStruct((8 * NP,), jnp.float32),
    mesh=_MESH,
    scratch_types=[
        pltpu.VMEM_SHARED((NP,), jnp.float32),
        pltpu.VMEM_SHARED((NP,), jnp.float32),
        pltpu.VMEM_SHARED((NP,), jnp.float32),
        pltpu.VMEM_SHARED((NP,), jnp.float32),
        pltpu.VMEM((3584,), jnp.int32),
        pltpu.VMEM((3584,), jnp.int32),
        pltpu.VMEM((3584,), jnp.int32),
        pltpu.VMEM((3584,), jnp.float32),
        pltpu.VMEM((3584,), jnp.float32),
        pltpu.VMEM((3584,), jnp.float32),
        pltpu.VMEM((3584,), jnp.float32),
        pltpu.VMEM((3584,), jnp.float32),
        pltpu.VMEM((3584,), jnp.float32),
        pltpu.VMEM((3584,), jnp.float32),
        pltpu.VMEM((3200,), jnp.float32),
        pltpu.SemaphoreType.DMA,
        pltpu.SemaphoreType.DMA,
        pltpu.SemaphoreType.DMA,
    ],
)
def _sc_layer1(s0e, d0e, ew0e, s1e, d1e, ew1e, xnx0, xny0, xnx1, xny1,
               agg_out, txs, tys, ax, ay, sidx, didxA, didxB, ewb, gx, gy,
               mxA, myA, mxB, myB, zbuf, semL, semG, semC):
    _pass_b(s0e, d0e, ew0e, s1e, d1e, ew1e, xnx0, xny0, xnx1, xny1,
            agg_out, txs, tys, ax, ay, sidx, didxA, didxB, ewb, gx, gy,
            mxA, myA, mxB, myB, zbuf, semL, semG, semC)


def _pass_c(s0e, d0e, s1e, d1e, h1n0, h1n1,
            agg_out,
            acc, sidx0, didx0, rows0, sidx1, didx1, rows1,
            sidx2, didx2, rows2, semL, semG, semC):
    c = lax.axis_index("c")
    s = lax.axis_index("s")
    w = s * 2 + c
    sets = ((sidx0, didx0, rows0), (sidx1, didx1, rows1),
            (sidx2, didx2, rows2))

    def zrows(i, _):
        rows0[i] = jnp.zeros((16,), jnp.float32)
        return 0

    for r, (se, de, tab) in enumerate(((s0e, d0e, h1n0), (s1e, d1e, h1n1))):
        lax.fori_loop(0, 512, zrows, 0)
        for i in range(12):
            pltpu.sync_copy(rows0, acc.at[pl.ds(s * 6400 + i * 512, 512), :])
        pltpu.sync_copy(rows0.at[pl.ds(0, 256), :],
                        acc.at[pl.ds(s * 6400 + 6144, 256), :])
        plsc.subcore_barrier()

        def body(j, _):
            base0 = (w * 98 + j * 14) * 512

            def load(i, t):
                si, di, _ = sets[t]
                l0 = pltpu.async_copy(
                    se.at[pl.ds(base0 + i * 512, 512)], si, semL)
                l1 = pltpu.async_copy(
                    de.at[pl.ds(base0 + i * 512, 512)], di, semL)
                l0.wait()
                l1.wait()

            def fire_gather(t):
                si, _, ro = sets[t]
                return pltpu.async_copy(tab.at[si], ro, semG)

            def fire_scatter(t):
                _, di, ro = sets[t]
                return pltpu.async_copy(ro, acc.at[di], semC, add=True)

            load(0, 0)
            gths = {0: fire_gather(0)}
            load(1, 1)
            gths[1] = fire_gather(1)
            scts = {}
            for i in range(14):
                t = i % 3
                gths[i].wait()
                scts[i] = fire_scatter(t)
                if i + 2 <= 13:
                    if i - 1 >= 0:
                        scts[i - 1].wait()
                    load(i + 2, (i + 2) % 3)
                    gths[i + 2] = fire_gather((i + 2) % 3)
            scts[11].wait()
            scts[12].wait()
            scts[13].wait()
            return 0

        lax.fori_loop(0, 7, body, 0)
        plsc.subcore_barrier()
        off = (c * 2 + r) * NP + s * 6400
        pltpu.sync_copy(acc.at[pl.ds(s * 6400, 6400), :],
                        agg_out.at[pl.ds(off, 6400), :])
        plsc.subcore_barrier()


@functools.partial(
    pl.kernel,
    out_type=jax.ShapeDtypeStruct((4 * NP, 16), jnp.float32),
    mesh=_MESH,
    compiler_params=pltpu.CompilerParams(use_tc_tiling_on_sc=False),
    scratch_types=[
        pltpu.VMEM_SHARED((NP, 16), jnp.float32),
        pltpu.VMEM((512,), jnp.int32),
        pltpu.VMEM((512,), jnp.int32),
        pltpu.VMEM((512, 16), jnp.float32),
        pltpu.VMEM((512,), jnp.int32),
        pltpu.VMEM((512,), jnp.int32),
        pltpu.VMEM((512, 16), jnp.float32),
        pltpu.VMEM((512,), jnp.int32),
        pltpu.VMEM((512,), jnp.int32),
        pltpu.VMEM((512, 16), jnp.float32),
        pltpu.SemaphoreType.DMA,
        pltpu.SemaphoreType.DMA,
        pltpu.SemaphoreType.DMA,
    ],
)
def _sc_layer2(s0e, d0e, s1e, d1e, h1n0, h1n1, agg_out,
               acc, sidx0, didx0, rows0, sidx1, didx1, rows1,
               sidx2, didx2, rows2, semL, semG, semC):
    _pass_c(s0e, d0e, s1e, d1e, h1n0, h1n1, agg_out,
            acc, sidx0, didx0, rows0, sidx1, didx1, rows1,
            sidx2, didx2, rows2, semL, semG, semC)


_HP = jax.lax.Precision.HIGHEST
_GRID = NP // 1024


def _t1_call(degs, xT):
    def body(d_ref, x_ref, xp_ref, odi_ref, idi_ref):
        d = d_ref[...]
        ds = d[0] + d[1]
        od0 = jax.lax.rsqrt(jnp.maximum(ds[0], 1.0))
        id0 = jax.lax.rsqrt(jnp.maximum(ds[1], 1.0))
        od1 = jax.lax.rsqrt(jnp.maximum(ds[2], 1.0))
        id1 = jax.lax.rsqrt(jnp.maximum(ds[3], 1.0))
        xv = x_ref[...]
        xp_ref[0] = xv[0] * od0
        xp_ref[1] = xv[1] * od0
        xp_ref[2] = xv[0] * od1
        xp_ref[3] = xv[1] * od1
        odi_ref[0] = od0
        odi_ref[1] = od1
        idi_ref[0] = id0
        idi_ref[1] = id1

    R = NP // 128
    return pl.pallas_call(
        body,
        grid=(R // 8,),
        in_specs=[
            pl.BlockSpec((2, 4, 8, 128), lambda i: (0, 0, i, 0)),
            pl.BlockSpec((2, 8, 128), lambda i: (0, i, 0)),
        ],
        out_specs=[
            pl.BlockSpec((4, 8, 128), lambda i: (0, i, 0)),
            pl.BlockSpec((2, 8, 128), lambda i: (0, i, 0)),
            pl.BlockSpec((2, 8, 128), lambda i: (0, i, 0)),
        ],
        out_shape=[
            jax.ShapeDtypeStruct((4, R, 128), jnp.float32),
            jax.ShapeDtypeStruct((2, R, 128), jnp.float32),
            jax.ShapeDtypeStruct((2, R, 128), jnp.float32),
        ],
    )(degs.reshape(2, 4, R, 128), xT.reshape(2, R, 128))


def _t2_call(aggB, idi, odi, Wcat, bsum):
    def body(a_ref, idi_ref, odi_ref, w_ref, b_ref, o0_ref, o1_ref):
        a = a_ref[...]
        asum = a[0] + a[1]
        idiv = idi_ref[...]
        a4 = asum * jnp.stack(
            [idiv[0], idiv[0], idiv[1], idiv[1]])
        dn = (((0,), (0,)), ((), ()))
        hT = jax.lax.dot_general(w_ref[...], a4, dn, precision=_HP)
        hT = jnp.maximum(hT + b_ref[...], 0.0)
        odiv = odi_ref[...]
        o0_ref[...] = jax.lax.transpose(hT * odiv[0:1, :], (1, 0))
        o1_ref[...] = jax.lax.transpose(hT * odiv[1:2, :], (1, 0))

    return pl.pallas_call(
        body,
        grid=(_GRID,),
        in_specs=[
            pl.BlockSpec((2, 4, 1024), lambda i: (0, 0, i)),
            pl.BlockSpec((2, 1024), lambda i: (0, i)),
            pl.BlockSpec((2, 1024), lambda i: (0, i)),
            pl.BlockSpec((4, 16), lambda i: (0, 0)),
            pl.BlockSpec((16, 1), lambda i: (0, 0)),
        ],
        out_specs=[
            pl.BlockSpec((1024, 16), lambda i: (i, 0)),
            pl.BlockSpec((1024, 16), lambda i: (i, 0)),
        ],
        out_shape=[
            jax.ShapeDtypeStruct((NP, 16), jnp.float32),
            jax.ShapeDtypeStruct((NP, 16), jnp.float32),
        ],
    )(aggB.reshape(2, 4, NP), idi.reshape(2, NP), odi.reshape(2, NP),
      Wcat, bsum.reshape(16, 1))


def _t3_call(agg2, idi, gcol, cnth, W2_0, W2_1, b2sum, Wc, bc):
    def body(p_ref, idi_ref, g_ref, cnt_ref, w20_ref, w21_ref, b2_ref,
             wc_ref, bc_ref, o_ref, acc):
        i = pl.program_id(0)

        @pl.when(i == 0)
        def _():
            acc[...] = jnp.zeros((256, 32), jnp.float32)

        p = p_ref[...]
        idiv = idi_ref[...]
        i0 = jax.lax.transpose(idiv[0:1, :], (1, 0))
        i1 = jax.lax.transpose(idiv[1:2, :], (1, 0))
        m0 = (p[0, 0] + p[1, 0]) * i0
        m1 = (p[0, 1] + p[1, 1]) * i1
        rhs = jnp.concatenate([m0, m1], axis=1).astype(jnp.bfloat16)
        gids = g_ref[...]
        onehot = (gids == jax.lax.broadcasted_iota(
            jnp.int32, (1, 256), 1)).astype(jnp.bfloat16)
        dn = (((0,), (0,)), ((), ()))
        acc[...] += jax.lax.dot_general(
            onehot, rhs, dn, preferred_element_type=jnp.float32)

        @pl.when(i == _GRID - 1)
        def _():
            cnt = cnt_ref[0, :256] + cnt_ref[1, :256]
            invc = (1.0 / jnp.maximum(cnt, 1.0))[:, None]
            maskg = (cnt >= 1.0).astype(jnp.float32)
            av = acc[...]
            hg = (jnp.dot(av[:, :16] * invc, w20_ref[...], precision=_HP)
                  + jnp.dot(av[:, 16:] * invc, w21_ref[...], precision=_HP)
                  + maskg[:, None] * b2_ref[...])
            o_ref[...] = jnp.dot(hg, wc_ref[...], precision=_HP) + bc_ref[...]

    return pl.pallas_call(
        body,
        grid=(_GRID,),
        in_specs=[
            pl.BlockSpec((2, 2, 1024, 16), lambda i: (0, 0, i, 0)),
            pl.BlockSpec((2, 1024), lambda i: (0, i)),
            pl.BlockSpec((1024, 1), lambda i: (i, 0)),
            pl.BlockSpec((2, 512), lambda i: (0, 0)),
            pl.BlockSpec((16, 16), lambda i: (0, 0)),
            pl.BlockSpec((16, 16), lambda i: (0, 0)),
            pl.BlockSpec((1, 16), lambda i: (0, 0)),
            pl.BlockSpec((16, 2), lambda i: (0, 0)),
            pl.BlockSpec((1, 2), lambda i: (0, 0)),
        ],
        out_specs=pl.BlockSpec((256, 2), lambda i: (0, 0)),
        out_shape=jax.ShapeDtypeStruct((256, 2), jnp.float32),
        scratch_shapes=[
            pltpu.VMEM((256, 32), jnp.float32),
        ],
    )(agg2, idi.reshape(2, NP), gcol, cnth, W2_0, W2_1,
      b2sum.reshape(1, 16), Wc, bc.reshape(1, 2))


def kernel(x, edge_index_r0, edge_weight_r0, edge_index_r1, edge_weight_r1,
           graph_ids, W1_r0, b1_r0, W1_r1, b1_r1, W2_r0, b2_r0, W2_r1,
           b2_r1, Wc, bc):
    s0 = edge_index_r0[0].astype(jnp.int32)
    d0 = edge_index_r0[1].astype(jnp.int32)
    s1 = edge_index_r1[0].astype(jnp.int32)
    d1 = edge_index_r1[1].astype(jnp.int32)
    g = graph_ids.astype(jnp.int32)
    gpad = jnp.concatenate([g, jnp.full((NPG - N,), 256, jnp.int32)])
    epad = jnp.full((EP - E,), N, jnp.int32)
    ewpad = jnp.zeros((EP - E,), jnp.float32)
    s0e = jnp.concatenate([s0, epad])
    d0e = jnp.concatenate([d0, epad])
    s1e = jnp.concatenate([s1, epad])
    d1e = jnp.concatenate([d1, epad])
    ew0e = jnp.concatenate([edge_weight_r0, ewpad])
    ew1e = jnp.concatenate([edge_weight_r1, ewpad])

    degs_flat, cnth_flat = _sc_hist(s0e, d0e, s1e, d1e, gpad)
    cnth = cnth_flat.reshape(2, 512)

    xT = jnp.pad(x.T, ((0, 0), (0, NP - N)))
    xplanes, odi, idi = _t1_call(degs_flat.reshape(2, 4, NP), xT)
    xp = xplanes.reshape(4, NP)

    aggB = _sc_layer1(s0e, d0e, ew0e, s1e, d1e, ew1e,
                      xp[0], xp[1], xp[2], xp[3])

    Wcat = jnp.concatenate([W1_r0, W1_r1], axis=0)
    h1n0, h1n1 = _t2_call(aggB, idi, odi, Wcat, b1_r0 + b1_r1)

    agg2 = _sc_layer2(s0e, d0e, s1e, d1e, h1n0, h1n1)
    agg2 = agg2.reshape(2, 2, NP, 16)

    gcol = jnp.concatenate(
        [g, jnp.full((NP - N,), 256, jnp.int32)]).reshape(NP, 1)
    return _t3_call(agg2, idi, gcol, cnth, W2_r0, W2_r1,
                    b2_r0 + b2_r1, Wc, bc)


# speedup vs baseline: 24.8297x; 1.2216x over previous
import functools

import jax
import jax.numpy as jnp
from jax import lax
from jax.experimental import pallas as pl
from jax.experimental.pallas import tpu as pltpu
from jax.experimental.pallas import tpu_sc as plsc

N = 100000
E = 1600000
B = 256
NP = 102400
EP = 1605632
GRP = EP // 1024
GPT = GRP // 32
NPG = 131072

_MESH = plsc.VectorSubcoreMesh(core_axis_name="c", subcore_axis_name="s")


def _zero_fill(buf, n16):
    z = jnp.zeros((16,), jnp.float32)

    def body(i, _):
        buf[pl.ds(i * 16, 16)] = z
        return 0

    lax.fori_loop(0, n16, body, 0)


def _fill_sentinel(buf, lo, n16):
    sent = jnp.full((16,), N, jnp.int32)
    for i in range(n16):
        buf[pl.ds(lo + i * 16, 16)] = sent


def _pass_a(ei0, ei1, gv,
            degs_out, cnth_out,
            h0, h1, h2, h3, cnth,
            bufsA0, bufsA1, bufsA2, bufsA3,
            bufsB0, bufsB1, bufsB2, bufsB3,
            ones, zbuf, semL, semS):
    c = lax.axis_index("c")
    s = lax.axis_index("s")
    w = s * 2 + c
    ept = E // 32
    bases = (0, E, 0, E)
    srcs = (ei0, ei0, ei1, ei1)
    hs = (h0, h1, h2, h3)
    bufsA = (bufsA0, bufsA1, bufsA2, bufsA3)
    bufsB = (bufsB0, bufsB1, bufsB2, bufsB3)

    _zero_fill(zbuf, 200)
    for i in range(64):
        ones[pl.ds(i * 16, 16)] = jnp.ones((16,), jnp.float32)

    for h in hs:
        pltpu.sync_copy(zbuf, h.at[pl.ds(s * 6400, 3200)])
        pltpu.sync_copy(zbuf, h.at[pl.ds(s * 6400 + 3200, 3200)])

    @pl.when(s == 0)
    def _():
        pltpu.sync_copy(zbuf.at[pl.ds(0, 512)], cnth)

    plsc.subcore_barrier()

    def one_group(off, cnt_e, bufs):
        ls = [pltpu.async_copy(
            srcs[i].at[pl.ds(bases[i] + off, cnt_e)],
            bufs[i] if cnt_e == 1024 else bufs[i].at[pl.ds(0, cnt_e)],
            semL) for i in range(4)]
        for l in ls:
            l.wait()
        return [pltpu.async_copy(ones, hs[i].at[bufs[i]], semS, add=True)
                for i in range(4)]

    def pair(k, _):
        off = w * ept + 2 * k * 1024
        cA = one_group(off, 1024, bufsA)
        cB = one_group(off + 1024, 1024, bufsB)
        for cp in cA + cB:
            cp.wait()
        return 0

    lax.fori_loop(0, 24, pair, 0)

    for i in range(4):
        _fill_sentinel(bufsA[i], 848, 11)
    for cp in one_group(w * ept + 48 * 1024, 848, bufsA):
        cp.wait()

    for i in range(4):
        pltpu.sync_copy(gv.at[pl.ds((w * 4 + i) * 1024, 1024)], bufsB[i])
    for cp in [pltpu.async_copy(ones, cnth.at[bufsB[i]], semS, add=True)
               for i in range(4)]:
        cp.wait()

    plsc.subcore_barrier()

    for t, h in enumerate(hs):
        off = (c * 4 + t) * NP + s * 6400
        pltpu.sync_copy(h.at[pl.ds(s * 6400, 6400)],
                        degs_out.at[pl.ds(off, 6400)])

    @pl.when(s == 0)
    def _():
        pltpu.sync_copy(cnth, cnth_out.at[pl.ds(c * 512, 512)])


@functools.partial(
    pl.kernel,
    out_type=[jax.ShapeDtypeStruct((8 * NP,), jnp.float32),
              jax.ShapeDtypeStruct((1024,), jnp.float32)],
    mesh=_MESH,
    scratch_types=[
        pltpu.VMEM_SHARED((NP,), jnp.float32),
        pltpu.VMEM_SHARED((NP,), jnp.float32),
        pltpu.VMEM_SHARED((NP,), jnp.float32),
        pltpu.VMEM_SHARED((NP,), jnp.float32),
        pltpu.VMEM_SHARED((512,), jnp.float32),
        pltpu.VMEM((1024,), jnp.int32),
        pltpu.VMEM((1024,), jnp.int32),
        pltpu.VMEM((1024,), jnp.int32),
        pltpu.VMEM((1024,), jnp.int32),
        pltpu.VMEM((1024,), jnp.int32),
        pltpu.VMEM((1024,), jnp.int32),
        pltpu.VMEM((1024,), jnp.int32),
        pltpu.VMEM((1024,), jnp.int32),
        pltpu.VMEM((1024,), jnp.float32),
        pltpu.VMEM((3200,), jnp.float32),
        pltpu.SemaphoreType.DMA,
        pltpu.SemaphoreType.DMA,
    ],
)
def _sc_hist(ei0, ei1, gv, degs_out, cnth_out,
             h0, h1, h2, h3, cnth,
             bufsA0, bufsA1, bufsA2, bufsA3,
             bufsB0, bufsB1, bufsB2, bufsB3,
             ones, zbuf, semL, semS):
    _pass_a(ei0, ei1, gv, degs_out, cnth_out,
            h0, h1, h2, h3, cnth,
            bufsA0, bufsA1, bufsA2, bufsA3,
            bufsB0, bufsB1, bufsB2, bufsB3,
            ones, zbuf, semL, semS)


def _pass_b(ei0, ew0, ei1, ew1, xnx0, xny0, xnx1, xny1,
            agg_out,
            txs, tys, ax, ay, sidx, didxA, didxB, ewb, gx, gy,
            mxA, myA, mxB, myB, zbuf, semL, semG, semC):
    c = lax.axis_index("c")
    s = lax.axis_index("s")
    w = s * 2 + c
    GB = 3584
    ept = E // 32

    _zero_fill(zbuf, 200)

    for r, (ei, ewe, tx, ty) in enumerate(
            ((ei0, ew0, xnx0, xny0), (ei1, ew1, xnx1, xny1))):
        pltpu.sync_copy(tx.at[pl.ds(s * 6400, 6400)],
                        txs.at[pl.ds(s * 6400, 6400)])
        pltpu.sync_copy(ty.at[pl.ds(s * 6400, 6400)],
                        tys.at[pl.ds(s * 6400, 6400)])
        pltpu.sync_copy(zbuf, ax.at[pl.ds(s * 6400, 3200)])
        pltpu.sync_copy(zbuf, ax.at[pl.ds(s * 6400 + 3200, 3200)])
        pltpu.sync_copy(zbuf, ay.at[pl.ds(s * 6400, 3200)])
        pltpu.sync_copy(zbuf, ay.at[pl.ds(s * 6400 + 3200, 3200)])
        plsc.subcore_barrier()

        def one_group(base, didx, mx, my, cnt=GB):
            sd = sidx if cnt == GB else sidx.at[pl.ds(0, cnt)]
            dd = didx if cnt == GB else didx.at[pl.ds(0, cnt)]
            ed = ewb if cnt == GB else ewb.at[pl.ds(0, cnt)]
            l0 = pltpu.async_copy(ei.at[pl.ds(base, cnt)], sd, semL)
            l1 = pltpu.async_copy(ei.at[pl.ds(E + base, cnt)], dd, semL)
            l2 = pltpu.async_copy(ewe.at[pl.ds(base, cnt)], ed, semL)
            l0.wait()
            l1.wait()
            l2.wait()
            if cnt != GB:
                _fill_sentinel(didx, cnt, (GB - cnt) // 16)
            g0 = pltpu.async_copy(txs.at[sidx], gx, semG)
            g1 = pltpu.async_copy(tys.at[sidx], gy, semG)
            g0.wait()
            g1.wait()

            def mul(l, _):
                for u in range(4):
                    sl = pl.ds((l * 4 + u) * 16, 16)
                    ew16 = ewb[sl]
                    mx[sl] = gx[sl] * ew16
                    my[sl] = gy[sl] * ew16
                return 0

            lax.fori_loop(0, GB // 64, mul, 0)
            c0 = pltpu.async_copy(mx, ax.at[didx], semC, add=True)
            c1 = pltpu.async_copy(my, ay.at[didx], semC, add=True)
            return c0, c1

        def pair(j, _):
            base = w * ept + 2 * j * GB
            cA0, cA1 = one_group(base, didxA, mxA, myA)
            cB0, cB1 = one_group(base + GB, didxB, mxB, myB)
            cA0.wait()
            cA1.wait()
            cB0.wait()
            cB1.wait()
            return 0

        lax.fori_loop(0, 6, pair, 0)
        cA0, cA1 = one_group(w * ept + 12 * GB, didxA, mxA, myA)
        cB0, cB1 = one_group(w * ept + 13 * GB, didxB, mxB, myB, cnt=3408)
        cA0.wait()
        cA1.wait()
        cB0.wait()
        cB1.wait()
        plsc.subcore_barrier()

        for p, acc in ((0, ax), (1, ay)):
            off = (c * 4 + r * 2 + p) * NP + s * 6400
            pltpu.sync_copy(acc.at[pl.ds(s * 6400, 6400)],
                            agg_out.at[pl.ds(off, 6400)])
        plsc.subcore_barrier()


@functools.partial(
    pl.kernel,
    out_type=jax.ShapeDtypeStruct((8 * NP,), jnp.float32),
    mesh=_MESH,
    scratch_types=[
        pltpu.VMEM_SHARED((NP,), jnp.float32),
        pltpu.VMEM_SHARED((NP,), jnp.float32),
        pltpu.VMEM_SHARED((NP,), jnp.float32),
        pltpu.VMEM_SHARED((NP,), jnp.float32),
        pltpu.VMEM((3584,), jnp.int32),
        pltpu.VMEM((3584,), jnp.int32),
        pltpu.VMEM((3584,), jnp.int32),
        pltpu.VMEM((3584,), jnp.float32),
        pltpu.VMEM((3584,), jnp.float32),
        pltpu.VMEM((3584,), jnp.float32),
        pltpu.VMEM((3584,), jnp.float32),
        pltpu.VMEM((3584,), jnp.float32),
        pltpu.VMEM((3584,), jnp.float32),
        pltpu.VMEM((3584,), jnp.float32),
        pltpu.VMEM((3200,), jnp.float32),
        pltpu.SemaphoreType.DMA,
        pltpu.SemaphoreType.DMA,
        pltpu.SemaphoreType.DMA,
    ],
)
def _sc_layer1(ei0, ew0, ei1, ew1, xnx0, xny0, xnx1, xny1,
               agg_out, txs, tys, ax, ay, sidx, didxA, didxB, ewb, gx, gy,
               mxA, myA, mxB, myB, zbuf, semL, semG, semC):
    _pass_b(ei0, ew0, ei1, ew1, xnx0, xny0, xnx1, xny1,
            agg_out, txs, tys, ax, ay, sidx, didxA, didxB, ewb, gx, gy,
            mxA, myA, mxB, myB, zbuf, semL, semG, semC)


def _pass_c(ei0, ei1, h1n0, h1n1,
            agg_out,
            acc, sidx0, didx0, rows0, sidx1, didx1, rows1,
            sidx2, didx2, rows2, semL, semG, semC):
    c = lax.axis_index("c")
    s = lax.axis_index("s")
    w = s * 2 + c
    ept = E // 32
    sets = ((sidx0, didx0, rows0), (sidx1, didx1, rows1),
            (sidx2, didx2, rows2))

    def zrows(i, _):
        rows0[i] = jnp.zeros((16,), jnp.float32)
        return 0

    for r, (ei, tab) in enumerate(((ei0, h1n0), (ei1, h1n1))):
        lax.fori_loop(0, 512, zrows, 0)
        for i in range(12):
            pltpu.sync_copy(rows0, acc.at[pl.ds(s * 6400 + i * 512, 512), :])
        pltpu.sync_copy(rows0.at[pl.ds(0, 256), :],
                        acc.at[pl.ds(s * 6400 + 6144, 256), :])
        plsc.subcore_barrier()

        def run_body(jbase, tail):
            def load(i, t, cnt=512):
                si, di, _ = sets[t]
                sd = si if cnt == 512 else si.at[pl.ds(0, cnt)]
                dd = di if cnt == 512 else di.at[pl.ds(0, cnt)]
                l0 = pltpu.async_copy(
                    ei.at[pl.ds(w * ept + (jbase + i) * 512, cnt)], sd, semL)
                l1 = pltpu.async_copy(
                    ei.at[pl.ds(E + w * ept + (jbase + i) * 512, cnt)],
                    dd, semL)
                l0.wait()
                l1.wait()
                if cnt != 512:
                    _fill_sentinel(di, cnt, (512 - cnt) // 16)

            def fire_gather(t):
                si, _, ro = sets[t]
                return pltpu.async_copy(tab.at[si], ro, semG)

            def fire_scatter(t):
                _, di, ro = sets[t]
                return pltpu.async_copy(ro, acc.at[di], semC, add=True)

            load(0, 0)
            gths = {0: fire_gather(0)}
            load(1, 1)
            gths[1] = fire_gather(1)
            scts = {}
            for i in range(14):
                t = i % 3
                gths[i].wait()
                scts[i] = fire_scatter(t)
                if i + 2 <= 13:
                    if i - 1 >= 0:
                        scts[i - 1].wait()
                    tset = (i + 2) % 3
                    if tail and i + 2 == 13:
                        load(i + 2, tset, cnt=336)
                    else:
                        load(i + 2, tset)
                    gths[i + 2] = fire_gather(tset)
            scts[11].wait()
            scts[12].wait()
            scts[13].wait()

        def body(j, _):
            run_body(j * 14, False)
            return 0

        lax.fori_loop(0, 6, body, 0)
        run_body(84, True)
        plsc.subcore_barrier()
        off = (c * 2 + r) * NP + s * 6400
        pltpu.sync_copy(acc.at[pl.ds(s * 6400, 6400), :],
                        agg_out.at[pl.ds(off, 6400), :])
        plsc.subcore_barrier()


@functools.partial(
    pl.kernel,
    out_type=jax.ShapeDtypeStruct((4 * NP, 16), jnp.float32),
    mesh=_MESH,
    compiler_params=pltpu.CompilerParams(use_tc_tiling_on_sc=False),
    scratch_types=[
        pltpu.VMEM_SHARED((NP, 16), jnp.float32),
        pltpu.VMEM((512,), jnp.int32),
        pltpu.VMEM((512,), jnp.int32),
        pltpu.VMEM((512, 16), jnp.float32),
        pltpu.VMEM((512,), jnp.int32),
        pltpu.VMEM((512,), jnp.int32),
        pltpu.VMEM((512, 16), jnp.float32),
        pltpu.VMEM((512,), jnp.int32),
        pltpu.VMEM((512,), jnp.int32),
        pltpu.VMEM((512, 16), jnp.float32),
        pltpu.SemaphoreType.DMA,
        pltpu.SemaphoreType.DMA,
        pltpu.SemaphoreType.DMA,
    ],
)
def _sc_layer2(ei0, ei1, h1n0, h1n1, agg_out,
               acc, sidx0, didx0, rows0, sidx1, didx1, rows1,
               sidx2, didx2, rows2, semL, semG, semC):
    _pass_c(ei0, ei1, h1n0, h1n1, agg_out,
            acc, sidx0, didx0, rows0, sidx1, didx1, rows1,
            sidx2, didx2, rows2, semL, semG, semC)


_HP = jax.lax.Precision.HIGHEST
_GRID = NP // 1024


def _t1_call(degs, xT):
    def body(d_ref, x_ref, xp_ref, odi_ref, idi_ref):
        d = d_ref[...]
        ds = d[0] + d[1]
        od0 = jax.lax.rsqrt(jnp.maximum(ds[0], 1.0))
        id0 = jax.lax.rsqrt(jnp.maximum(ds[1], 1.0))
        od1 = jax.lax.rsqrt(jnp.maximum(ds[2], 1.0))
        id1 = jax.lax.rsqrt(jnp.maximum(ds[3], 1.0))
        xv = x_ref[...]
        xp_ref[0] = xv[0] * od0
        xp_ref[1] = xv[1] * od0
        xp_ref[2] = xv[0] * od1
        xp_ref[3] = xv[1] * od1
        odi_ref[0] = od0
        odi_ref[1] = od1
        idi_ref[0] = id0
        idi_ref[1] = id1

    R = NP // 128
    return pl.pallas_call(
        body,
        grid=(R // 8,),
        in_specs=[
            pl.BlockSpec((2, 4, 8, 128), lambda i: (0, 0, i, 0)),
            pl.BlockSpec((2, 8, 128), lambda i: (0, i, 0)),
        ],
        out_specs=[
            pl.BlockSpec((4, 8, 128), lambda i: (0, i, 0)),
            pl.BlockSpec((2, 8, 128), lambda i: (0, i, 0)),
            pl.BlockSpec((2, 8, 128), lambda i: (0, i, 0)),
        ],
        out_shape=[
            jax.ShapeDtypeStruct((4, R, 128), jnp.float32),
            jax.ShapeDtypeStruct((2, R, 128), jnp.float32),
            jax.ShapeDtypeStruct((2, R, 128), jnp.float32),
        ],
    )(degs.reshape(2, 4, R, 128), xT.reshape(2, R, 128))


def _t2_call(aggB, idi, odi, Wcat, bsum):
    def body(a_ref, idi_ref, odi_ref, w_ref, b_ref, o0_ref, o1_ref):
        a = a_ref[...]
        asum = a[0] + a[1]
        idiv = idi_ref[...]
        a4 = asum * jnp.stack(
            [idiv[0], idiv[0], idiv[1], idiv[1]])
        dn = (((0,), (0,)), ((), ()))
        hT = jax.lax.dot_general(w_ref[...], a4, dn, precision=_HP)
        hT = jnp.maximum(hT + b_ref[...], 0.0)
        odiv = odi_ref[...]
        o0_ref[...] = jax.lax.transpose(hT * odiv[0:1, :], (1, 0))
        o1_ref[...] = jax.lax.transpose(hT * odiv[1:2, :], (1, 0))

    return pl.pallas_call(
        body,
        grid=(_GRID,),
        in_specs=[
            pl.BlockSpec((2, 4, 1024), lambda i: (0, 0, i)),
            pl.BlockSpec((2, 1024), lambda i: (0, i)),
            pl.BlockSpec((2, 1024), lambda i: (0, i)),
            pl.BlockSpec((4, 16), lambda i: (0, 0)),
            pl.BlockSpec((16, 1), lambda i: (0, 0)),
        ],
        out_specs=[
            pl.BlockSpec((1024, 16), lambda i: (i, 0)),
            pl.BlockSpec((1024, 16), lambda i: (i, 0)),
        ],
        out_shape=[
            jax.ShapeDtypeStruct((NP, 16), jnp.float32),
            jax.ShapeDtypeStruct((NP, 16), jnp.float32),
        ],
    )(aggB.reshape(2, 4, NP), idi.reshape(2, NP), odi.reshape(2, NP),
      Wcat, bsum.reshape(16, 1))


def _t3_call(agg2, idi, gcol, cnth, W2_0, W2_1, b2sum, Wc, bc):
    def body(p_ref, idi_ref, g_ref, cnt_ref, w20_ref, w21_ref, b2_ref,
             wc_ref, bc_ref, o_ref, acc):
        i = pl.program_id(0)

        @pl.when(i == 0)
        def _():
            acc[...] = jnp.zeros((256, 32), jnp.float32)

        p = p_ref[...]
        idiv = idi_ref[...]
        i0 = jax.lax.transpose(idiv[0:1, :], (1, 0))
        i1 = jax.lax.transpose(idiv[1:2, :], (1, 0))
        m0 = (p[0, 0] + p[1, 0]) * i0
        m1 = (p[0, 1] + p[1, 1]) * i1
        rhs = jnp.concatenate([m0, m1], axis=1).astype(jnp.bfloat16)
        gids = g_ref[...]
        onehot = (gids == jax.lax.broadcasted_iota(
            jnp.int32, (1, 256), 1)).astype(jnp.bfloat16)
        dn = (((0,), (0,)), ((), ()))
        acc[...] += jax.lax.dot_general(
            onehot, rhs, dn, preferred_element_type=jnp.float32)

        @pl.when(i == _GRID - 1)
        def _():
            cnt = cnt_ref[0, :256] + cnt_ref[1, :256]
            invc = (1.0 / jnp.maximum(cnt, 1.0))[:, None]
            maskg = (cnt >= 1.0).astype(jnp.float32)
            av = acc[...]
            hg = (jnp.dot(av[:, :16] * invc, w20_ref[...], precision=_HP)
                  + jnp.dot(av[:, 16:] * invc, w21_ref[...], precision=_HP)
                  + maskg[:, None] * b2_ref[...])
            o_ref[...] = jnp.dot(hg, wc_ref[...], precision=_HP) + bc_ref[...]

    return pl.pallas_call(
        body,
        grid=(_GRID,),
        in_specs=[
            pl.BlockSpec((2, 2, 1024, 16), lambda i: (0, 0, i, 0)),
            pl.BlockSpec((2, 1024), lambda i: (0, i)),
            pl.BlockSpec((1024, 1), lambda i: (i, 0)),
            pl.BlockSpec((2, 512), lambda i: (0, 0)),
            pl.BlockSpec((16, 16), lambda i: (0, 0)),
            pl.BlockSpec((16, 16), lambda i: (0, 0)),
            pl.BlockSpec((1, 16), lambda i: (0, 0)),
            pl.BlockSpec((16, 2), lambda i: (0, 0)),
            pl.BlockSpec((1, 2), lambda i: (0, 0)),
        ],
        out_specs=pl.BlockSpec((256, 2), lambda i: (0, 0)),
        out_shape=jax.ShapeDtypeStruct((256, 2), jnp.float32),
        scratch_shapes=[
            pltpu.VMEM((256, 32), jnp.float32),
        ],
    )(agg2, idi.reshape(2, NP), gcol, cnth, W2_0, W2_1,
      b2sum.reshape(1, 16), Wc, bc.reshape(1, 2))


def kernel(x, edge_index_r0, edge_weight_r0, edge_index_r1, edge_weight_r1,
           graph_ids, W1_r0, b1_r0, W1_r1, b1_r1, W2_r0, b2_r0, W2_r1,
           b2_r1, Wc, bc):
    g = graph_ids.astype(jnp.int32)
    gpad = jnp.concatenate([g, jnp.full((NPG - N,), 256, jnp.int32)])
    ei0f = edge_index_r0.astype(jnp.int32).reshape(2 * E)
    ei1f = edge_index_r1.astype(jnp.int32).reshape(2 * E)

    degs_flat, cnth_flat = _sc_hist(ei0f, ei1f, gpad)
    cnth = cnth_flat.reshape(2, 512)

    xT = jnp.pad(x.T, ((0, 0), (0, NP - N)))
    xplanes, odi, idi = _t1_call(degs_flat.reshape(2, 4, NP), xT)
    xp = xplanes.reshape(4, NP)

    aggB = _sc_layer1(ei0f, edge_weight_r0, ei1f, edge_weight_r1,
                      xp[0], xp[1], xp[2], xp[3])

    Wcat = jnp.concatenate([W1_r0, W1_r1], axis=0)
    h1n0, h1n1 = _t2_call(aggB, idi, odi, Wcat, b1_r0 + b1_r1)

    agg2 = _sc_layer2(ei0f, ei1f, h1n0, h1n1)
    agg2 = agg2.reshape(2, 2, NP, 16)

    gcol = jnp.concatenate(
        [g, jnp.full((NP - N,), 256, jnp.int32)]).reshape(NP, 1)
    return _t3_call(agg2, idi, gcol, cnth, W2_r0, W2_r1,
                    b2_r0 + b2_r1, Wc, bc)


# speedup vs baseline: 28.6184x; 1.1526x over previous
import functools

import jax
import jax.numpy as jnp
from jax import lax
from jax.experimental import pallas as pl
from jax.experimental.pallas import tpu as pltpu
from jax.experimental.pallas import tpu_sc as plsc

N = 100000
E = 1600000
B = 256
NP = 102400
EP = 1605632
GRP = EP // 1024
GPT = GRP // 32
NPG = 131072

_MESH = plsc.VectorSubcoreMesh(core_axis_name="c", subcore_axis_name="s")


def _zero_fill(buf, n16):
    z = jnp.zeros((16,), jnp.float32)

    def body(i, _):
        buf[pl.ds(i * 16, 16)] = z
        return 0

    lax.fori_loop(0, n16, body, 0)


def _fill_sentinel(buf, lo, n16):
    sent = jnp.full((16,), N, jnp.int32)
    for i in range(n16):
        buf[pl.ds(lo + i * 16, 16)] = sent


def _pass_a(ei0, ei1, gv,
            degs_out, cnth_out,
            h0, h1, h2, h3, cnth,
            bufsA0, bufsA1, bufsA2, bufsA3,
            bufsB0, bufsB1, bufsB2, bufsB3,
            ones, zbuf, semL, semS):
    c = lax.axis_index("c")
    s = lax.axis_index("s")
    w = s * 2 + c
    ept = E // 32
    bases = (0, E, 0, E)
    srcs = (ei0, ei0, ei1, ei1)
    hs = (h0, h1, h2, h3)
    bufsA = (bufsA0, bufsA1, bufsA2, bufsA3)
    bufsB = (bufsB0, bufsB1, bufsB2, bufsB3)

    _zero_fill(zbuf, 200)
    for i in range(64):
        ones[pl.ds(i * 16, 16)] = jnp.ones((16,), jnp.float32)

    for h in hs:
        pltpu.sync_copy(zbuf, h.at[pl.ds(s * 6400, 3200)])
        pltpu.sync_copy(zbuf, h.at[pl.ds(s * 6400 + 3200, 3200)])

    @pl.when(s == 0)
    def _():
        pltpu.sync_copy(zbuf.at[pl.ds(0, 512)], cnth)

    plsc.subcore_barrier()

    def one_group(off, cnt_e, bufs):
        ls = [pltpu.async_copy(
            srcs[i].at[pl.ds(bases[i] + off, cnt_e)],
            bufs[i] if cnt_e == 1024 else bufs[i].at[pl.ds(0, cnt_e)],
            semL) for i in range(4)]
        for l in ls:
            l.wait()
        return [pltpu.async_copy(ones, hs[i].at[bufs[i]], semS, add=True)
                for i in range(4)]

    def pair(k, _):
        off = w * ept + 2 * k * 1024
        cA = one_group(off, 1024, bufsA)
        cB = one_group(off + 1024, 1024, bufsB)
        for cp in cA + cB:
            cp.wait()
        return 0

    lax.fori_loop(0, 24, pair, 0)

    for i in range(4):
        _fill_sentinel(bufsA[i], 848, 11)
    for cp in one_group(w * ept + 48 * 1024, 848, bufsA):
        cp.wait()

    for i in range(4):
        pltpu.sync_copy(gv.at[pl.ds((w * 4 + i) * 1024, 1024)], bufsB[i])
    for cp in [pltpu.async_copy(ones, cnth.at[bufsB[i]], semS, add=True)
               for i in range(4)]:
        cp.wait()

    plsc.subcore_barrier()

    for t, h in enumerate(hs):
        off = (c * 4 + t) * NP + s * 6400
        pltpu.sync_copy(h.at[pl.ds(s * 6400, 6400)],
                        degs_out.at[pl.ds(off, 6400)])

    @pl.when(s == 0)
    def _():
        pltpu.sync_copy(cnth, cnth_out.at[pl.ds(c * 512, 512)])


@functools.partial(
    pl.kernel,
    out_type=[jax.ShapeDtypeStruct((8 * NP,), jnp.float32),
              jax.ShapeDtypeStruct((1024,), jnp.float32)],
    mesh=_MESH,
    scratch_types=[
        pltpu.VMEM_SHARED((NP,), jnp.float32),
        pltpu.VMEM_SHARED((NP,), jnp.float32),
        pltpu.VMEM_SHARED((NP,), jnp.float32),
        pltpu.VMEM_SHARED((NP,), jnp.float32),
        pltpu.VMEM_SHARED((512,), jnp.float32),
        pltpu.VMEM((1024,), jnp.int32),
        pltpu.VMEM((1024,), jnp.int32),
        pltpu.VMEM((1024,), jnp.int32),
        pltpu.VMEM((1024,), jnp.int32),
        pltpu.VMEM((1024,), jnp.int32),
        pltpu.VMEM((1024,), jnp.int32),
        pltpu.VMEM((1024,), jnp.int32),
        pltpu.VMEM((1024,), jnp.int32),
        pltpu.VMEM((1024,), jnp.float32),
        pltpu.VMEM((3200,), jnp.float32),
        pltpu.SemaphoreType.DMA,
        pltpu.SemaphoreType.DMA,
    ],
)
def _sc_hist(ei0, ei1, gv, degs_out, cnth_out,
             h0, h1, h2, h3, cnth,
             bufsA0, bufsA1, bufsA2, bufsA3,
             bufsB0, bufsB1, bufsB2, bufsB3,
             ones, zbuf, semL, semS):
    _pass_a(ei0, ei1, gv, degs_out, cnth_out,
            h0, h1, h2, h3, cnth,
            bufsA0, bufsA1, bufsA2, bufsA3,
            bufsB0, bufsB1, bufsB2, bufsB3,
            ones, zbuf, semL, semS)


def _pass_b(ei0, ew0, ei1, ew1, xnx0, xny0, xnx1, xny1,
            agg_out,
            txs, tys, ax, ay, sidx, didxA, didxB, ewb, gx, gy,
            mxA, myA, mxB, myB, zbuf, semL, semG, semC):
    c = lax.axis_index("c")
    s = lax.axis_index("s")
    w = s * 2 + c
    GB = 3584
    ept = E // 32

    _zero_fill(zbuf, 200)

    for r, (ei, ewe, tx, ty) in enumerate(
            ((ei0, ew0, xnx0, xny0), (ei1, ew1, xnx1, xny1))):
        pltpu.sync_copy(tx.at[pl.ds(s * 6400, 6400)],
                        txs.at[pl.ds(s * 6400, 6400)])
        pltpu.sync_copy(ty.at[pl.ds(s * 6400, 6400)],
                        tys.at[pl.ds(s * 6400, 6400)])
        pltpu.sync_copy(zbuf, ax.at[pl.ds(s * 6400, 3200)])
        pltpu.sync_copy(zbuf, ax.at[pl.ds(s * 6400 + 3200, 3200)])
        pltpu.sync_copy(zbuf, ay.at[pl.ds(s * 6400, 3200)])
        pltpu.sync_copy(zbuf, ay.at[pl.ds(s * 6400 + 3200, 3200)])
        plsc.subcore_barrier()

        def one_group(base, didx, mx, my, cnt=GB):
            sd = sidx if cnt == GB else sidx.at[pl.ds(0, cnt)]
            dd = didx if cnt == GB else didx.at[pl.ds(0, cnt)]
            ed = ewb if cnt == GB else ewb.at[pl.ds(0, cnt)]
            l0 = pltpu.async_copy(ei.at[pl.ds(base, cnt)], sd, semL)
            l1 = pltpu.async_copy(ei.at[pl.ds(E + base, cnt)], dd, semL)
            l2 = pltpu.async_copy(ewe.at[pl.ds(base, cnt)], ed, semL)
            l0.wait()
            l1.wait()
            l2.wait()
            if cnt != GB:
                _fill_sentinel(didx, cnt, (GB - cnt) // 16)
            g0 = pltpu.async_copy(txs.at[sidx], gx, semG)
            g1 = pltpu.async_copy(tys.at[sidx], gy, semG)
            g0.wait()
            g1.wait()

            def mul(l, _):
                for u in range(4):
                    sl = pl.ds((l * 4 + u) * 16, 16)
                    ew16 = ewb[sl]
                    mx[sl] = gx[sl] * ew16
                    my[sl] = gy[sl] * ew16
                return 0

            lax.fori_loop(0, GB // 64, mul, 0)
            c0 = pltpu.async_copy(mx, ax.at[didx], semC, add=True)
            c1 = pltpu.async_copy(my, ay.at[didx], semC, add=True)
            return c0, c1

        def pair(j, _):
            base = w * ept + 2 * j * GB
            cA0, cA1 = one_group(base, didxA, mxA, myA)
            cB0, cB1 = one_group(base + GB, didxB, mxB, myB)
            cA0.wait()
            cA1.wait()
            cB0.wait()
            cB1.wait()
            return 0

        lax.fori_loop(0, 6, pair, 0)
        cA0, cA1 = one_group(w * ept + 12 * GB, didxA, mxA, myA)
        cB0, cB1 = one_group(w * ept + 13 * GB, didxB, mxB, myB, cnt=3408)
        cA0.wait()
        cA1.wait()
        cB0.wait()
        cB1.wait()
        plsc.subcore_barrier()

        for p, acc in ((0, ax), (1, ay)):
            off = (c * 4 + r * 2 + p) * NP + s * 6400
            pltpu.sync_copy(acc.at[pl.ds(s * 6400, 6400)],
                            agg_out.at[pl.ds(off, 6400)])
        plsc.subcore_barrier()


@functools.partial(
    pl.kernel,
    out_type=jax.ShapeDtypeStruct((8 * NP,), jnp.float32),
    mesh=_MESH,
    scratch_types=[
        pltpu.VMEM_SHARED((NP,), jnp.float32),
        pltpu.VMEM_SHARED((NP,), jnp.float32),
        pltpu.VMEM_SHARED((NP,), jnp.float32),
        pltpu.VMEM_SHARED((NP,), jnp.float32),
        pltpu.VMEM((3584,), jnp.int32),
        pltpu.VMEM((3584,), jnp.int32),
        pltpu.VMEM((3584,), jnp.int32),
        pltpu.VMEM((3584,), jnp.float32),
        pltpu.VMEM((3584,), jnp.float32),
        pltpu.VMEM((3584,), jnp.float32),
        pltpu.VMEM((3584,), jnp.float32),
        pltpu.VMEM((3584,), jnp.float32),
        pltpu.VMEM((3584,), jnp.float32),
        pltpu.VMEM((3584,), jnp.float32),
        pltpu.VMEM((3200,), jnp.float32),
        pltpu.SemaphoreType.DMA,
        pltpu.SemaphoreType.DMA,
        pltpu.SemaphoreType.DMA,
    ],
)
def _sc_layer1(ei0, ew0, ei1, ew1, xnx0, xny0, xnx1, xny1,
               agg_out, txs, tys, ax, ay, sidx, didxA, didxB, ewb, gx, gy,
               mxA, myA, mxB, myB, zbuf, semL, semG, semC):
    _pass_b(ei0, ew0, ei1, ew1, xnx0, xny0, xnx1, xny1,
            agg_out, txs, tys, ax, ay, sidx, didxA, didxB, ewb, gx, gy,
            mxA, myA, mxB, myB, zbuf, semL, semG, semC)


def _pass_c(ei0, ei1, h1n0, h1n1, idif, gn,
            pool_out,
            acc, sidx0, didx0, rows0, sidx1, didx1, rows1,
            sidx2, didx2, rows2, idib, semL, semG, semC):
    c = lax.axis_index("c")
    s = lax.axis_index("s")
    w = s * 2 + c
    ept = E // 32
    sets = ((sidx0, didx0, rows0), (sidx1, didx1, rows1),
            (sidx2, didx2, rows2))
    i16 = lax.iota(jnp.int32, 16)

    def zbuf2d(buf):
        def zr(i, _):
            buf[i] = jnp.zeros((16,), jnp.float32)
            return 0

        lax.fori_loop(0, 512, zr, 0)

    for r, (ei, tab) in enumerate(((ei0, h1n0), (ei1, h1n1))):
        zbuf2d(rows0)
        for i in range(12):
            pltpu.sync_copy(rows0, acc.at[pl.ds(s * 6400 + i * 512, 512), :])
        pltpu.sync_copy(rows0.at[pl.ds(0, 256), :],
                        acc.at[pl.ds(s * 6400 + 6144, 256), :])
        plsc.subcore_barrier()

        def run_body(jbase, tail):
            def load(i, t, cnt=512):
                si, di, _ = sets[t]
                sd = si if cnt == 512 else si.at[pl.ds(0, cnt)]
                dd = di if cnt == 512 else di.at[pl.ds(0, cnt)]
                l0 = pltpu.async_copy(
                    ei.at[pl.ds(w * ept + (jbase + i) * 512, cnt)], sd, semL)
                l1 = pltpu.async_copy(
                    ei.at[pl.ds(E + w * ept + (jbase + i) * 512, cnt)],
                    dd, semL)
                l0.wait()
                l1.wait()
                if cnt != 512:
                    _fill_sentinel(di, cnt, (512 - cnt) // 16)

            def fire_gather(t):
                si, _, ro = sets[t]
                return pltpu.async_copy(tab.at[si], ro, semG)

            def fire_scatter(t):
                _, di, ro = sets[t]
                return pltpu.async_copy(ro, acc.at[di], semC, add=True)

            load(0, 0)
            gths = {0: fire_gather(0)}
            load(1, 1)
            gths[1] = fire_gather(1)
            scts = {}
            for i in range(14):
                t = i % 3
                gths[i].wait()
                scts[i] = fire_scatter(t)
                if i + 2 <= 13:
                    if i - 1 >= 0:
                        scts[i - 1].wait()
                    tset = (i + 2) % 3
                    if tail and i + 2 == 13:
                        load(i + 2, tset, cnt=336)
                    else:
                        load(i + 2, tset)
                    gths[i + 2] = fire_gather(tset)
            scts[11].wait()
            scts[12].wait()
            scts[13].wait()

        def body(j, _):
            run_body(j * 14, False)
            return 0

        lax.fori_loop(0, 6, body, 0)
        run_body(84, True)
        plsc.subcore_barrier()

        zbuf2d(rows2)
        for ch in range(13):
            cnt = 512 if ch < 12 else 256
            base = s * 6400 + ch * 512
            rsl = rows0 if cnt == 512 else rows0.at[pl.ds(0, cnt), :]
            pltpu.sync_copy(acc.at[pl.ds(base, cnt), :], rsl)
            isl = idib if cnt == 512 else idib.at[pl.ds(0, cnt)]
            pltpu.sync_copy(idif.at[pl.ds(r * NP + base, cnt)], isl)
            gsl = sidx1 if cnt == 512 else sidx1.at[pl.ds(0, cnt)]
            pltpu.sync_copy(gn.at[pl.ds(base, cnt)], gsl)

            def node(j, _):
                bc = i16 * 0 + j
                gj = plsc.load_gather(sidx1, [bc])
                ij = plsc.load_gather(idib, [bc])
                val = rows0[j] * ij
                plsc.addupdate_scatter(rows2, [gj, i16], val)
                return 0

            lax.fori_loop(0, cnt, node, 0)

        plsc.subcore_barrier()
        zbuf2d(rows0)

        @pl.when(s == 0)
        def _():
            pltpu.sync_copy(rows0, acc.at[pl.ds(0, 512), :])

        for i in range(32):
            sidx0[pl.ds(i * 16, 16)] = i16 + i * 16
        plsc.subcore_barrier()
        pltpu.async_copy(rows2, acc.at[sidx0], semC, add=True).wait()
        plsc.subcore_barrier()

        @pl.when(s == 0)
        def _():
            pltpu.sync_copy(acc.at[pl.ds(0, 264), :],
                            pool_out.at[pl.ds((c * 2 + r) * 264, 264), :])
        plsc.subcore_barrier()


@functools.partial(
    pl.kernel,
    out_type=jax.ShapeDtypeStruct((4 * 264, 16), jnp.float32),
    mesh=_MESH,
    compiler_params=pltpu.CompilerParams(use_tc_tiling_on_sc=False,
                                         needs_layout_passes=False),
    scratch_types=[
        pltpu.VMEM_SHARED((NP, 16), jnp.float32),
        pltpu.VMEM((512,), jnp.int32),
        pltpu.VMEM((512,), jnp.int32),
        pltpu.VMEM((512, 16), jnp.float32),
        pltpu.VMEM((512,), jnp.int32),
        pltpu.VMEM((512,), jnp.int32),
        pltpu.VMEM((512, 16), jnp.float32),
        pltpu.VMEM((512,), jnp.int32),
        pltpu.VMEM((512,), jnp.int32),
        pltpu.VMEM((512, 16), jnp.float32),
        pltpu.VMEM((512,), jnp.float32),
        pltpu.SemaphoreType.DMA,
        pltpu.SemaphoreType.DMA,
        pltpu.SemaphoreType.DMA,
    ],
)
def _sc_layer2(ei0, ei1, h1n0, h1n1, idif, gn, pool_out,
               acc, sidx0, didx0, rows0, sidx1, didx1, rows1,
               sidx2, didx2, rows2, idib, semL, semG, semC):
    _pass_c(ei0, ei1, h1n0, h1n1, idif, gn, pool_out,
            acc, sidx0, didx0, rows0, sidx1, didx1, rows1,
            sidx2, didx2, rows2, idib, semL, semG, semC)


_HP = jax.lax.Precision.HIGHEST
_GRID = NP // 1024


def _t1_call(degs, xT):
    def body(d_ref, x_ref, xp_ref, odi_ref, idi_ref):
        d = d_ref[...]
        ds = d[0] + d[1]
        od0 = jax.lax.rsqrt(jnp.maximum(ds[0], 1.0))
        id0 = jax.lax.rsqrt(jnp.maximum(ds[1], 1.0))
        od1 = jax.lax.rsqrt(jnp.maximum(ds[2], 1.0))
        id1 = jax.lax.rsqrt(jnp.maximum(ds[3], 1.0))
        xv = x_ref[...]
        xp_ref[0] = xv[0] * od0
        xp_ref[1] = xv[1] * od0
        xp_ref[2] = xv[0] * od1
        xp_ref[3] = xv[1] * od1
        odi_ref[0] = od0
        odi_ref[1] = od1
        idi_ref[0] = id0
        idi_ref[1] = id1

    R = NP // 128
    return pl.pallas_call(
        body,
        grid=(R // 8,),
        in_specs=[
            pl.BlockSpec((2, 4, 8, 128), lambda i: (0, 0, i, 0)),
            pl.BlockSpec((2, 8, 128), lambda i: (0, i, 0)),
        ],
        out_specs=[
            pl.BlockSpec((4, 8, 128), lambda i: (0, i, 0)),
            pl.BlockSpec((2, 8, 128), lambda i: (0, i, 0)),
            pl.BlockSpec((2, 8, 128), lambda i: (0, i, 0)),
        ],
        out_shape=[
            jax.ShapeDtypeStruct((4, R, 128), jnp.float32),
            jax.ShapeDtypeStruct((2, R, 128), jnp.float32),
            jax.ShapeDtypeStruct((2, R, 128), jnp.float32),
        ],
    )(degs.reshape(2, 4, R, 128), xT.reshape(2, R, 128))


def _t2_call(aggB, idi, odi, Wcat, bsum):
    def body(a_ref, idi_ref, odi_ref, w_ref, b_ref, o0_ref, o1_ref):
        a = a_ref[...]
        asum = a[0] + a[1]
        idiv = idi_ref[...]
        a4 = asum * jnp.stack(
            [idiv[0], idiv[0], idiv[1], idiv[1]])
        dn = (((0,), (0,)), ((), ()))
        hT = jax.lax.dot_general(w_ref[...], a4, dn, precision=_HP)
        hT = jnp.maximum(hT + b_ref[...], 0.0)
        odiv = odi_ref[...]
        o0_ref[...] = jax.lax.transpose(hT * odiv[0:1, :], (1, 0))
        o1_ref[...] = jax.lax.transpose(hT * odiv[1:2, :], (1, 0))

    return pl.pallas_call(
        body,
        grid=(_GRID,),
        in_specs=[
            pl.BlockSpec((2, 4, 1024), lambda i: (0, 0, i)),
            pl.BlockSpec((2, 1024), lambda i: (0, i)),
            pl.BlockSpec((2, 1024), lambda i: (0, i)),
            pl.BlockSpec((4, 16), lambda i: (0, 0)),
            pl.BlockSpec((16, 1), lambda i: (0, 0)),
        ],
        out_specs=[
            pl.BlockSpec((1024, 16), lambda i: (i, 0)),
            pl.BlockSpec((1024, 16), lambda i: (i, 0)),
        ],
        out_shape=[
            jax.ShapeDtypeStruct((NP, 16), jnp.float32),
            jax.ShapeDtypeStruct((NP, 16), jnp.float32),
        ],
    )(aggB.reshape(2, 4, NP), idi.reshape(2, NP), odi.reshape(2, NP),
      Wcat, bsum.reshape(16, 1))


def _t3_call(pool, cnth, W2_0, W2_1, b2sum, Wc, bc):
    def body(p_ref, cnt_ref, w20_ref, w21_ref, b2_ref, wc_ref, bc_ref,
             o_ref):
        p = p_ref[...]
        cnt = cnt_ref[0, :256] + cnt_ref[1, :256]
        invc = (1.0 / jnp.maximum(cnt, 1.0))[:, None]
        maskg = (cnt >= 1.0).astype(jnp.float32)
        p0 = (p[0, 0, :256, :] + p[1, 0, :256, :]) * invc
        p1 = (p[0, 1, :256, :] + p[1, 1, :256, :]) * invc
        hg = (jnp.dot(p0, w20_ref[...], precision=_HP)
              + jnp.dot(p1, w21_ref[...], precision=_HP)
              + maskg[:, None] * b2_ref[...])
        o_ref[...] = jnp.dot(hg, wc_ref[...], precision=_HP) + bc_ref[...]

    return pl.pallas_call(
        body,
        grid=(1,),
        in_specs=[
            pl.BlockSpec((2, 2, 264, 16), lambda i: (0, 0, 0, 0)),
            pl.BlockSpec((2, 512), lambda i: (0, 0)),
            pl.BlockSpec((16, 16), lambda i: (0, 0)),
            pl.BlockSpec((16, 16), lambda i: (0, 0)),
            pl.BlockSpec((1, 16), lambda i: (0, 0)),
            pl.BlockSpec((16, 2), lambda i: (0, 0)),
            pl.BlockSpec((1, 2), lambda i: (0, 0)),
        ],
        out_specs=pl.BlockSpec((256, 2), lambda i: (0, 0)),
        out_shape=jax.ShapeDtypeStruct((256, 2), jnp.float32),
    )(pool, cnth, W2_0, W2_1, b2sum.reshape(1, 16), Wc, bc.reshape(1, 2))


def kernel(x, edge_index_r0, edge_weight_r0, edge_index_r1, edge_weight_r1,
           graph_ids, W1_r0, b1_r0, W1_r1, b1_r1, W2_r0, b2_r0, W2_r1,
           b2_r1, Wc, bc):
    g = graph_ids.astype(jnp.int32)
    gpad = jnp.concatenate([g, jnp.full((NPG - N,), 256, jnp.int32)])
    ei0f = edge_index_r0.astype(jnp.int32).reshape(2 * E)
    ei1f = edge_index_r1.astype(jnp.int32).reshape(2 * E)

    degs_flat, cnth_flat = _sc_hist(ei0f, ei1f, gpad)
    cnth = cnth_flat.reshape(2, 512)

    xT = jnp.pad(x.T, ((0, 0), (0, NP - N)))
    xplanes, odi, idi = _t1_call(degs_flat.reshape(2, 4, NP), xT)
    xp = xplanes.reshape(4, NP)

    aggB = _sc_layer1(ei0f, edge_weight_r0, ei1f, edge_weight_r1,
                      xp[0], xp[1], xp[2], xp[3])

    Wcat = jnp.concatenate([W1_r0, W1_r1], axis=0)
    h1n0, h1n1 = _t2_call(aggB, idi, odi, Wcat, b1_r0 + b1_r1)

    gn = jnp.concatenate([g, jnp.full((NP - N,), 256, jnp.int32)])
    pool = _sc_layer2(ei0f, ei1f, h1n0, h1n1, idi.reshape(2 * NP), gn)

    return _t3_call(pool.reshape(2, 2, 264, 16), cnth, W2_r0, W2_r1,
                    b2_r0 + b2_r1, Wc, bc)
